# baseline (speedup 1.0000x reference)
import dataclasses
import functools

import jax
import jax.numpy as jnp
from jax import lax
from jax.experimental import pallas as pl
from jax.experimental.pallas import tpu as pltpu
from jax.experimental.pallas import tpu_sc as plsc

N, E, D, R, H = 10000, 320000, 128, 38, 4
HD = D // H
RPAD = 40

NC, NS, L = 2, 16, 16
NW = NC * NS
EW = E // NW
B = 80
NBLK = EW // B
B4 = 40
NBLK4 = EW // B4
NR = N // NS
ZR = 25
NZ = NR // ZR

_mesh = plsc.VectorSubcoreMesh(core_axis_name="c", subcore_axis_name="s")

_cp = pltpu.CompilerParams()
_fields = pltpu.CompilerParams.__dataclass_fields__
if "needs_layout_passes" in _fields:
    _cp = dataclasses.replace(_cp, needs_layout_passes=False)
if "use_tc_tiling_on_sc" in _fields:
    _cp = dataclasses.replace(_cp, use_tc_tiling_on_sc=False)


def _f32(shape):
    return jax.ShapeDtypeStruct(shape, jnp.float32)


def _k1_body(x_ref, w_ref, asrc_ref, adst_ref, rel_ref, arel_ref,
             h_ref, st_ref, sr_ref):
    h = jnp.dot(x_ref[...], w_ref[...], preferred_element_type=jnp.float32)
    h_ref[...] = h
    ss = jnp.dot(h, asrc_ref[...], preferred_element_type=jnp.float32)
    sd = jnp.dot(h, adst_ref[...], preferred_element_type=jnp.float32)
    st_ref[...] = jnp.concatenate([ss, sd], axis=1)
    sr_ref[...] = jnp.dot(rel_ref[...], arel_ref[...],
                          preferred_element_type=jnp.float32)


_k1 = pl.pallas_call(
    _k1_body,
    out_shape=(_f32((N, D)), _f32((N, 8)), _f32((R, H))),
)


def _k2_body(src_hbm, dst_hbm, et_hbm, sctab_hbm, srel_hbm,
             ex_hbm, dpart_hbm,
             sctab_v, srel_v, src_v, dst_v, et_v, ex_v, expad_v, zbuf_v,
             dacc_sh):
    cid = lax.axis_index("c")
    sid = lax.axis_index("s")
    wid = cid * NS + sid

    pltpu.sync_copy(sctab_hbm, sctab_v)
    pltpu.sync_copy(srel_hbm, srel_v)

    zeros16 = jnp.zeros((L,), jnp.float32)

    @pl.loop(0, B)
    def _(r):
        expad_v[r, pl.ds(0, L)] = zeros16

    @pl.loop(0, ZR)
    def _(r):
        zbuf_v[r, pl.ds(0, L)] = zeros16

    @pl.loop(0, NZ)
    def _(k):
        pltpu.sync_copy(zbuf_v, dacc_sh.at[pl.ds(sid * NR + k * ZR, ZR)])

    plsc.subcore_barrier()

    lane = lax.iota(jnp.int32, L)

    @pl.loop(0, NBLK)
    def _(blk):
        off = wid * EW + blk * B
        pltpu.sync_copy(src_hbm.at[pl.ds(off, B)], src_v)
        pltpu.sync_copy(dst_hbm.at[pl.ds(off, B)], dst_v)
        pltpu.sync_copy(et_hbm.at[pl.ds(off, B)], et_v)

        @pl.loop(0, B, step=L)
        def _(g):
            srcv = src_v[pl.ds(g, L)]
            dstv = dst_v[pl.ds(g, L)]
            etv = et_v[pl.ds(g, L)]
            rows = g + lane
            bs = srcv * 8
            bd = dstv * 8 + 4
            br = etv * 4
            for hh in range(H):
                a = plsc.load_gather(sctab_v, [bs + hh])
                b = plsc.load_gather(sctab_v, [bd + hh])
                c = plsc.load_gather(srel_v, [br + hh])
                al = a + b + c
                al = jnp.maximum(al, al * 0.2)
                ev = jnp.exp(al)
                plsc.store_scatter(ex_v, [rows * 4 + hh], ev)
                plsc.store_scatter(
                    expad_v, [rows, jnp.full((L,), hh, jnp.int32)], ev)

        pltpu.sync_copy(ex_v, ex_hbm.at[pl.ds(off * 4, B * 4)])
        pltpu.sync_copy(expad_v, dacc_sh.at[dst_v], add=True)

    plsc.subcore_barrier()

    pltpu.sync_copy(dacc_sh.at[pl.ds(sid * NR, NR)],
                    dpart_hbm.at[cid, pl.ds(sid * NR, NR)])


@jax.jit
def _k2(src, dst, et, sctab, srel_pad):
    kfn = pl.kernel(
        _k2_body,
        out_type=(_f32((E * 4,)), _f32((NC, N, 16))),
        mesh=_mesh,
        scratch_types=[
            pltpu.VMEM((N * 8,), jnp.float32),
            pltpu.VMEM((RPAD * 4,), jnp.float32),
            pltpu.VMEM((B,), jnp.int32),
            pltpu.VMEM((B,), jnp.int32),
            pltpu.VMEM((B,), jnp.int32),
            pltpu.VMEM((B * 4,), jnp.float32),
            pltpu.VMEM((B, 16), jnp.float32),
            pltpu.VMEM((ZR, 16), jnp.float32),
            pltpu.VMEM_SHARED((N, 16), jnp.float32),
        ],
        compiler_params=_cp,
    )
    return kfn(src, dst, et, sctab.reshape(N * 8), srel_pad.reshape(RPAD * 4))


def _k3_body(dp_ref, invd_ref):
    d = dp_ref[0] + dp_ref[1]
    iv = 1.0 / (d + 1e-16)
    col = lax.broadcasted_iota(jnp.int32, (N, 16), 1)
    invd_ref[...] = jnp.where(col < H, iv, 0.0)


_k3 = pl.pallas_call(_k3_body, out_shape=_f32((N, 16)))


def _k4_body(src_hbm, dst_hbm, et_hbm, ex_hbm, invd_hbm, h_hbm, rel_hbm,
             attn_hbm, opart_hbm,
             src0, dst0, dsc0, et0, ex0, hsrc0, rel0, ivd0, attn0, msg0,
             src1, dst1, dsc1, et1, ex1, hsrc1, rel1, ivd1, attn1, msg1,
             zbuf_v, sem_in0, sem_in1, sem_g0, sem_g1,
             sem_a0, sem_a1, sem_s0, sem_s1, oacc_sh):
    cid = lax.axis_index("c")
    sid = lax.axis_index("s")
    wid = cid * NS + sid

    srcs = (src0, src1)
    dsts = (dst0, dst1)
    dscs = (dsc0, dsc1)
    ets = (et0, et1)
    exs = (ex0, ex1)
    hsrcs = (hsrc0, hsrc1)
    rels = (rel0, rel1)
    ivds = (ivd0, ivd1)
    attns = (attn0, attn1)
    msgs = (msg0, msg1)
    sem_in = (sem_in0, sem_in1)
    sem_g = (sem_g0, sem_g1)
    sem_a = (sem_a0, sem_a1)
    sem_s = (sem_s0, sem_s1)

    zeros16 = jnp.zeros((L,), jnp.float32)

    @pl.loop(0, ZR)
    def _(r):
        for cc in range(D // L):
            zbuf_v[r, pl.ds(cc * L, L)] = zeros16

    @pl.loop(0, NZ)
    def _(k):
        pltpu.sync_copy(zbuf_v, oacc_sh.at[pl.ds(sid * NR + k * ZR, ZR)])

    plsc.subcore_barrier()

    lane = lax.iota(jnp.int32, L)
    lane_d4 = lax.shift_right_logical(lane, 2)
    lane_m4 = jnp.bitwise_and(lane, 3)

    def in_copies(blk, p):
        off = wid * EW + blk * B4
        return (
            pltpu.make_async_copy(src_hbm.at[pl.ds(off, B4)], srcs[p],
                                  sem_in[p]),
            pltpu.make_async_copy(dst_hbm.at[pl.ds(off, B4)], dsts[p],
                                  sem_in[p]),
            pltpu.make_async_copy(et_hbm.at[pl.ds(off, B4)], ets[p],
                                  sem_in[p]),
            pltpu.make_async_copy(ex_hbm.at[pl.ds(off * 4, B4 * 4)], exs[p],
                                  sem_in[p]),
        )

    def g_copies(p):
        return (
            pltpu.make_async_copy(h_hbm.at[srcs[p]], hsrcs[p], sem_g[p]),
            pltpu.make_async_copy(rel_hbm.at[ets[p]], rels[p], sem_g[p]),
            pltpu.make_async_copy(invd_hbm.at[dsts[p]], ivds[p], sem_g[p]),
        )

    def attn_copy(blk, p):
        off = wid * EW + blk * B4
        return pltpu.make_async_copy(
            attns[p], attn_hbm.at[pl.ds(off * 4, B4 * 4)], sem_a[p])

    def issue_in(blk, p):
        for c in in_copies(blk, p):
            c.start()

    def wait_in(blk, p):
        for c in in_copies(blk, p):
            c.wait()

    def issue_g(p):
        for c in g_copies(p):
            c.start()

    def wait_g(p):
        for c in g_copies(p):
            c.wait()

    def drain_out(blk, p):
        @pl.when(blk >= 2)
        def _():
            attn_copy(blk - 2, p).wait()
            pltpu.make_async_copy(msgs[p], oacc_sh.at[dscs[p]],
                                  sem_s[p]).wait()

    def compute(blk, p):
        drain_out(blk, p)

        @pl.loop(0, B4 // 4)
        def _(g):
            iv = plsc.load_gather(ivds[p], [g * 4 + lane_d4, lane_m4])
            exv = exs[p][pl.ds(g * L, L)]
            attns[p][pl.ds(g * L, L)] = exv * iv

        attn_copy(blk, p).start()

        @pl.loop(0, B4)
        def _(e):
            for hh in range(H):
                sp = plsc.load_gather(
                    attns[p], [jnp.full((L,), e * 4 + hh, jnp.int32)])
                for cc in range(2):
                    col = hh * HD + cc * L
                    hv = hsrcs[p][e, pl.ds(col, L)]
                    rv = rels[p][e, pl.ds(col, L)]
                    msgs[p][e, pl.ds(col, L)] = (hv + rv) * sp

        for o in (0, 16, B4 - L):
            dscs[p][pl.ds(o, L)] = dsts[p][pl.ds(o, L)]
        pltpu.async_copy(msgs[p], oacc_sh.at[dscs[p]], sem_s[p], add=True)

    issue_in(0, 0)
    issue_in(1, 1)
    wait_in(0, 0)
    issue_g(0)

    @pl.loop(0, NBLK4 // 2)
    def _(it):
        e0 = it * 2
        wait_in(e0 + 1, 1)
        issue_g(1)
        wait_g(0)
        compute(e0, 0)

        @pl.when(e0 + 2 < NBLK4)
        def _():
            issue_in(e0 + 2, 0)
            wait_in(e0 + 2, 0)
            issue_g(0)

        wait_g(1)
        compute(e0 + 1, 1)

        @pl.when(e0 + 3 < NBLK4)
        def _():
            issue_in(e0 + 3, 1)

    attn_copy(NBLK4 - 2, 0).wait()
    pltpu.make_async_copy(msgs[0], oacc_sh.at[dscs[0]], sem_s[0]).wait()
    attn_copy(NBLK4 - 1, 1).wait()
    pltpu.make_async_copy(msgs[1], oacc_sh.at[dscs[1]], sem_s[1]).wait()

    plsc.subcore_barrier()

    pltpu.sync_copy(oacc_sh.at[pl.ds(sid * NR, NR)],
                    opart_hbm.at[cid, pl.ds(sid * NR, NR)])


@jax.jit
def _k4(src, dst, et, ex, invd, h, rel_emb):
    buf_set = [
        pltpu.VMEM((B4,), jnp.int32),
        pltpu.VMEM((B4,), jnp.int32),
        pltpu.VMEM((B4,), jnp.int32),
        pltpu.VMEM((B4,), jnp.int32),
        pltpu.VMEM((B4 * 4,), jnp.float32),
        pltpu.VMEM((B4, D), jnp.float32),
        pltpu.VMEM((B4, D), jnp.float32),
        pltpu.VMEM((B4, 16), jnp.float32),
        pltpu.VMEM((B4 * 4,), jnp.float32),
        pltpu.VMEM((B4, D), jnp.float32),
    ]
    kfn = pl.kernel(
        _k4_body,
        out_type=(_f32((E * 4,)), _f32((NC, N, D))),
        mesh=_mesh,
        scratch_types=(
            buf_set + buf_set
            + [
                pltpu.VMEM((ZR, D), jnp.float32),
                pltpu.SemaphoreType.DMA,
                pltpu.SemaphoreType.DMA,
                pltpu.SemaphoreType.DMA,
                pltpu.SemaphoreType.DMA,
                pltpu.SemaphoreType.DMA,
                pltpu.SemaphoreType.DMA,
                pltpu.SemaphoreType.DMA,
                pltpu.SemaphoreType.DMA,
                pltpu.VMEM_SHARED((N, D), jnp.float32),
            ]
        ),
        compiler_params=_cp,
    )
    return kfn(src, dst, et, ex, invd, h, rel_emb)


def _k5_body(op_ref, out_ref):
    out_ref[...] = jax.nn.gelu(op_ref[0] + op_ref[1])


_k5 = pl.pallas_call(_k5_body, out_shape=_f32((N, D)))


def _blockdiag(att):
    d = jnp.arange(D)
    return jnp.zeros((D, H), jnp.float32).at[d, d // HD].set(att.reshape(D))


@jax.jit
def kernel(x, edge_index, edge_type, W, rel_emb, att_src, att_dst, att_rel):
    src = edge_index[0]
    dst = edge_index[1]
    et = edge_type

    asrc = _blockdiag(att_src)
    adst = _blockdiag(att_dst)
    arel = _blockdiag(att_rel)

    h, sctab, srel = _k1(x, W, asrc, adst, rel_emb, arel)
    srel_pad = jnp.pad(srel, ((0, RPAD - R), (0, 0)))

    ex, dpart = _k2(src, dst, et, sctab, srel_pad)
    invd = _k3(dpart)
    attn_flat, opart = _k4(src, dst, et, ex, invd, h, rel_emb)
    out = _k5(opart)
    return out, attn_flat.reshape(E, H)


# speedup vs baseline: 1.0109x; 1.0109x over previous
import dataclasses
import functools

import jax
import jax.numpy as jnp
from jax import lax
from jax.experimental import pallas as pl
from jax.experimental.pallas import tpu as pltpu
from jax.experimental.pallas import tpu_sc as plsc

N, E, D, R, H = 10000, 320000, 128, 38, 4
HD = D // H
RPAD = 40

NC, NS, L = 2, 16, 16
NW = NC * NS
EW = E // NW
B = 80
NBLK = EW // B
B4 = 40
NBLK4 = EW // B4
NR = N // NS
ZR = 25
NZ = NR // ZR

_mesh = plsc.VectorSubcoreMesh(core_axis_name="c", subcore_axis_name="s")

_cp = pltpu.CompilerParams()
_fields = pltpu.CompilerParams.__dataclass_fields__
if "needs_layout_passes" in _fields:
    _cp = dataclasses.replace(_cp, needs_layout_passes=False)
if "use_tc_tiling_on_sc" in _fields:
    _cp = dataclasses.replace(_cp, use_tc_tiling_on_sc=False)


def _f32(shape):
    return jax.ShapeDtypeStruct(shape, jnp.float32)


def _k1_body(x_ref, w_ref, asrc_ref, adst_ref, rel_ref, arel_ref,
             h_ref, st_ref, sr_ref):
    h = jnp.dot(x_ref[...], w_ref[...], preferred_element_type=jnp.float32)
    h_ref[...] = h
    ss = jnp.dot(h, asrc_ref[...], preferred_element_type=jnp.float32)
    sd = jnp.dot(h, adst_ref[...], preferred_element_type=jnp.float32)
    st_ref[...] = jnp.concatenate([ss, sd], axis=1)
    sr_ref[...] = jnp.dot(rel_ref[...], arel_ref[...],
                          preferred_element_type=jnp.float32)


_k1 = pl.pallas_call(
    _k1_body,
    out_shape=(_f32((N, D)), _f32((N, 8)), _f32((R, H))),
)


def _k2_body(src_hbm, dst_hbm, et_hbm, sctab_hbm, srel_hbm,
             ex_hbm, dpart_hbm,
             sctab_v, srel_v, src_v, dst_v, et_v, ex_v, expad_v, zbuf_v,
             dacc_sh):
    cid = lax.axis_index("c")
    sid = lax.axis_index("s")
    wid = cid * NS + sid

    pltpu.sync_copy(sctab_hbm, sctab_v)
    pltpu.sync_copy(srel_hbm, srel_v)

    zeros16 = jnp.zeros((L,), jnp.float32)

    @pl.loop(0, B)
    def _(r):
        expad_v[r, pl.ds(0, L)] = zeros16

    @pl.loop(0, ZR)
    def _(r):
        zbuf_v[r, pl.ds(0, L)] = zeros16

    @pl.loop(0, NZ)
    def _(k):
        pltpu.sync_copy(zbuf_v, dacc_sh.at[pl.ds(sid * NR + k * ZR, ZR)])

    plsc.subcore_barrier()

    lane = lax.iota(jnp.int32, L)

    @pl.loop(0, NBLK)
    def _(blk):
        off = wid * EW + blk * B
        pltpu.sync_copy(src_hbm.at[pl.ds(off, B)], src_v)
        pltpu.sync_copy(dst_hbm.at[pl.ds(off, B)], dst_v)
        pltpu.sync_copy(et_hbm.at[pl.ds(off, B)], et_v)

        @pl.loop(0, B, step=L)
        def _(g):
            srcv = src_v[pl.ds(g, L)]
            dstv = dst_v[pl.ds(g, L)]
            etv = et_v[pl.ds(g, L)]
            rows = g + lane
            bs = srcv * 8
            bd = dstv * 8 + 4
            br = etv * 4
            for hh in range(H):
                a = plsc.load_gather(sctab_v, [bs + hh])
                b = plsc.load_gather(sctab_v, [bd + hh])
                c = plsc.load_gather(srel_v, [br + hh])
                al = a + b + c
                al = jnp.maximum(al, al * 0.2)
                ev = jnp.exp(al)
                plsc.store_scatter(ex_v, [rows * 4 + hh], ev)
                plsc.store_scatter(
                    expad_v, [rows, jnp.full((L,), hh, jnp.int32)], ev)

        pltpu.sync_copy(ex_v, ex_hbm.at[pl.ds(off * 4, B * 4)])
        pltpu.sync_copy(expad_v, dacc_sh.at[dst_v], add=True)

    plsc.subcore_barrier()

    pltpu.sync_copy(dacc_sh.at[pl.ds(sid * NR, NR)],
                    dpart_hbm.at[cid, pl.ds(sid * NR, NR)])


@jax.jit
def _k2(src, dst, et, sctab, srel_pad):
    kfn = pl.kernel(
        _k2_body,
        out_type=(_f32((E * 4,)), _f32((NC, N, 16))),
        mesh=_mesh,
        scratch_types=[
            pltpu.VMEM((N * 8,), jnp.float32),
            pltpu.VMEM((RPAD * 4,), jnp.float32),
            pltpu.VMEM((B,), jnp.int32),
            pltpu.VMEM((B,), jnp.int32),
            pltpu.VMEM((B,), jnp.int32),
            pltpu.VMEM((B * 4,), jnp.float32),
            pltpu.VMEM((B, 16), jnp.float32),
            pltpu.VMEM((ZR, 16), jnp.float32),
            pltpu.VMEM_SHARED((N, 16), jnp.float32),
        ],
        compiler_params=_cp,
    )
    return kfn(src, dst, et, sctab.reshape(N * 8), srel_pad.reshape(RPAD * 4))


def _k3_body(dp_ref, invd_ref):
    d = dp_ref[0] + dp_ref[1]
    iv = 1.0 / (d + 1e-16)
    col = lax.broadcasted_iota(jnp.int32, (N, 16), 1)
    invd_ref[...] = jnp.where(col < H, iv, 0.0)


_k3 = pl.pallas_call(_k3_body, out_shape=_f32((N, 16)))


def _k4_body(src_hbm, dst_hbm, et_hbm, ex_hbm, invd_hbm, h_hbm, rel_hbm,
             attn_hbm, opart_hbm,
             src0, dst0, dsc0, et0, ex0, hsrc0, rel0, ivd0, attn0, msg0,
             src1, dst1, dsc1, et1, ex1, hsrc1, rel1, ivd1, attn1, msg1,
             zbuf_v, sem_in0, sem_in1, sem_g0, sem_g1,
             sem_a0, sem_a1, sem_s0, sem_s1, oacc_sh):
    cid = lax.axis_index("c")
    sid = lax.axis_index("s")
    wid = cid * NS + sid

    srcs = (src0, src1)
    dsts = (dst0, dst1)
    dscs = (dsc0, dsc1)
    ets = (et0, et1)
    exs = (ex0, ex1)
    hsrcs = (hsrc0, hsrc1)
    rels = (rel0, rel1)
    ivds = (ivd0, ivd1)
    attns = (attn0, attn1)
    msgs = (msg0, msg1)
    sem_in = (sem_in0, sem_in1)
    sem_g = (sem_g0, sem_g1)
    sem_a = (sem_a0, sem_a1)
    sem_s = (sem_s0, sem_s1)

    zeros16 = jnp.zeros((L,), jnp.float32)

    @pl.loop(0, ZR)
    def _(r):
        for cc in range(D // L):
            zbuf_v[r, pl.ds(cc * L, L)] = zeros16

    @pl.loop(0, NZ)
    def _(k):
        pltpu.sync_copy(zbuf_v, oacc_sh.at[pl.ds(sid * NR + k * ZR, ZR)])

    plsc.subcore_barrier()

    lane = lax.iota(jnp.int32, L)
    lane_d4 = lax.shift_right_logical(lane, 2)
    lane_m4 = jnp.bitwise_and(lane, 3)

    def in_copies(blk, p):
        off = wid * EW + blk * B4
        return (
            pltpu.make_async_copy(src_hbm.at[pl.ds(off, B4)], srcs[p],
                                  sem_in[p]),
            pltpu.make_async_copy(dst_hbm.at[pl.ds(off, B4)], dsts[p],
                                  sem_in[p]),
            pltpu.make_async_copy(et_hbm.at[pl.ds(off, B4)], ets[p],
                                  sem_in[p]),
            pltpu.make_async_copy(ex_hbm.at[pl.ds(off * 4, B4 * 4)], exs[p],
                                  sem_in[p]),
        )

    def g_copies(p):
        return (
            pltpu.make_async_copy(h_hbm.at[srcs[p]], hsrcs[p], sem_g[p]),
            pltpu.make_async_copy(rel_hbm.at[ets[p]], rels[p], sem_g[p]),
            pltpu.make_async_copy(invd_hbm.at[dsts[p]], ivds[p], sem_g[p]),
        )

    def attn_copy(blk, p):
        off = wid * EW + blk * B4
        return pltpu.make_async_copy(
            attns[p], attn_hbm.at[pl.ds(off * 4, B4 * 4)], sem_a[p])

    def issue_in(blk, p):
        for c in in_copies(blk, p):
            c.start()

    def wait_in(blk, p):
        for c in in_copies(blk, p):
            c.wait()

    def issue_g(p):
        for c in g_copies(p):
            c.start()

    def wait_g(p):
        for c in g_copies(p):
            c.wait()

    def drain_out(blk, p):
        @pl.when(blk >= 2)
        def _():
            attn_copy(blk - 2, p).wait()
            pltpu.make_async_copy(msgs[p], oacc_sh.at[dscs[p]],
                                  sem_s[p]).wait()

    def compute(blk, p):
        drain_out(blk, p)

        @pl.loop(0, B4 // 4)
        def _(g):
            iv = plsc.load_gather(ivds[p], [g * 4 + lane_d4, lane_m4])
            exv = exs[p][pl.ds(g * L, L)]
            attns[p][pl.ds(g * L, L)] = exv * iv

        attn_copy(blk, p).start()

        @pl.loop(0, B4 // 4)
        def _(g):
            at16 = attns[p][pl.ds(g * L, L)]
            for el in range(4):
                e = g * 4 + el
                for hh in range(H):
                    sp = jnp.broadcast_to(at16[el * 4 + hh], (L,))
                    for cc in range(2):
                        col = hh * HD + cc * L
                        hv = hsrcs[p][e, pl.ds(col, L)]
                        rv = rels[p][e, pl.ds(col, L)]
                        msgs[p][e, pl.ds(col, L)] = (hv + rv) * sp

        for o in (0, 16, B4 - L):
            dscs[p][pl.ds(o, L)] = dsts[p][pl.ds(o, L)]
        pltpu.async_copy(msgs[p], oacc_sh.at[dscs[p]], sem_s[p], add=True)

    issue_in(0, 0)
    issue_in(1, 1)
    wait_in(0, 0)
    issue_g(0)

    @pl.loop(0, NBLK4 // 2)
    def _(it):
        e0 = it * 2
        wait_in(e0 + 1, 1)
        issue_g(1)
        wait_g(0)
        compute(e0, 0)

        @pl.when(e0 + 2 < NBLK4)
        def _():
            issue_in(e0 + 2, 0)
            wait_in(e0 + 2, 0)
            issue_g(0)

        wait_g(1)
        compute(e0 + 1, 1)

        @pl.when(e0 + 3 < NBLK4)
        def _():
            issue_in(e0 + 3, 1)

    attn_copy(NBLK4 - 2, 0).wait()
    pltpu.make_async_copy(msgs[0], oacc_sh.at[dscs[0]], sem_s[0]).wait()
    attn_copy(NBLK4 - 1, 1).wait()
    pltpu.make_async_copy(msgs[1], oacc_sh.at[dscs[1]], sem_s[1]).wait()

    plsc.subcore_barrier()

    pltpu.sync_copy(oacc_sh.at[pl.ds(sid * NR, NR)],
                    opart_hbm.at[cid, pl.ds(sid * NR, NR)])


@jax.jit
def _k4(src, dst, et, ex, invd, h, rel_emb):
    buf_set = [
        pltpu.VMEM((B4,), jnp.int32),
        pltpu.VMEM((B4,), jnp.int32),
        pltpu.VMEM((B4,), jnp.int32),
        pltpu.VMEM((B4,), jnp.int32),
        pltpu.VMEM((B4 * 4,), jnp.float32),
        pltpu.VMEM((B4, D), jnp.float32),
        pltpu.VMEM((B4, D), jnp.float32),
        pltpu.VMEM((B4, 16), jnp.float32),
        pltpu.VMEM((B4 * 4,), jnp.float32),
        pltpu.VMEM((B4, D), jnp.float32),
    ]
    kfn = pl.kernel(
        _k4_body,
        out_type=(_f32((E * 4,)), _f32((NC, N, D))),
        mesh=_mesh,
        scratch_types=(
            buf_set + buf_set
            + [
                pltpu.VMEM((ZR, D), jnp.float32),
                pltpu.SemaphoreType.DMA,
                pltpu.SemaphoreType.DMA,
                pltpu.SemaphoreType.DMA,
                pltpu.SemaphoreType.DMA,
                pltpu.SemaphoreType.DMA,
                pltpu.SemaphoreType.DMA,
                pltpu.SemaphoreType.DMA,
                pltpu.SemaphoreType.DMA,
                pltpu.VMEM_SHARED((N, D), jnp.float32),
            ]
        ),
        compiler_params=_cp,
    )
    return kfn(src, dst, et, ex, invd, h, rel_emb)


def _k5_body(op_ref, out_ref):
    out_ref[...] = jax.nn.gelu(op_ref[0] + op_ref[1])


_k5 = pl.pallas_call(_k5_body, out_shape=_f32((N, D)))


def _blockdiag(att):
    d = jnp.arange(D)
    return jnp.zeros((D, H), jnp.float32).at[d, d // HD].set(att.reshape(D))


@jax.jit
def kernel(x, edge_index, edge_type, W, rel_emb, att_src, att_dst, att_rel):
    src = edge_index[0]
    dst = edge_index[1]
    et = edge_type

    asrc = _blockdiag(att_src)
    adst = _blockdiag(att_dst)
    arel = _blockdiag(att_rel)

    h, sctab, srel = _k1(x, W, asrc, adst, rel_emb, arel)
    srel_pad = jnp.pad(srel, ((0, RPAD - R), (0, 0)))

    ex, dpart = _k2(src, dst, et, sctab, srel_pad)
    invd = _k3(dpart)
    attn_flat, opart = _k4(src, dst, et, ex, invd, h, rel_emb)
    out = _k5(opart)
    return out, attn_flat.reshape(E, H)


# speedup vs baseline: 1.0466x; 1.0353x over previous
import dataclasses
import functools

import jax
import jax.numpy as jnp
from jax import lax
from jax.experimental import pallas as pl
from jax.experimental.pallas import tpu as pltpu
from jax.experimental.pallas import tpu_sc as plsc

N, E, D, R, H = 10000, 320000, 128, 38, 4
HD = D // H
RPAD = 40

NC, NS, L = 2, 16, 16
NW = NC * NS
EW = E // NW
B = 80
NBLK = EW // B
B4 = 40
NBLK4 = EW // B4
NR = N // NS
ZR = 25
NZ = NR // ZR

_mesh = plsc.VectorSubcoreMesh(core_axis_name="c", subcore_axis_name="s")

_cp = pltpu.CompilerParams()
_fields = pltpu.CompilerParams.__dataclass_fields__
if "needs_layout_passes" in _fields:
    _cp = dataclasses.replace(_cp, needs_layout_passes=False)
if "use_tc_tiling_on_sc" in _fields:
    _cp = dataclasses.replace(_cp, use_tc_tiling_on_sc=False)


def _f32(shape):
    return jax.ShapeDtypeStruct(shape, jnp.float32)


def _k1_body(x_ref, w_ref, asrc_ref, adst_ref, rel_ref, arel_ref,
             h_ref, st_ref, sr_ref):
    h = jnp.dot(x_ref[...], w_ref[...], preferred_element_type=jnp.float32)
    h_ref[...] = h
    ss = jnp.dot(h, asrc_ref[...], preferred_element_type=jnp.float32)
    sd = jnp.dot(h, adst_ref[...], preferred_element_type=jnp.float32)
    st_ref[...] = jnp.concatenate([ss, sd], axis=1)
    sr_ref[...] = jnp.dot(rel_ref[...], arel_ref[...],
                          preferred_element_type=jnp.float32)


_k1 = pl.pallas_call(
    _k1_body,
    out_shape=(_f32((N, D)), _f32((N, 8)), _f32((R, H))),
)


def _k2_body(src_hbm, dst_hbm, et_hbm, sctab_hbm, srel_hbm,
             ex_hbm, dpart_hbm,
             sctab_v, srel_v, src_v, dst_v, et_v, ex_v, expad_v, zbuf_v,
             dacc_sh):
    cid = lax.axis_index("c")
    sid = lax.axis_index("s")
    wid = cid * NS + sid

    pltpu.sync_copy(sctab_hbm, sctab_v)
    pltpu.sync_copy(srel_hbm, srel_v)

    zeros16 = jnp.zeros((L,), jnp.float32)

    @pl.loop(0, B)
    def _(r):
        expad_v[r, pl.ds(0, L)] = zeros16

    @pl.loop(0, ZR)
    def _(r):
        zbuf_v[r, pl.ds(0, L)] = zeros16

    @pl.loop(0, NZ)
    def _(k):
        pltpu.sync_copy(zbuf_v, dacc_sh.at[pl.ds(sid * NR + k * ZR, ZR)])

    plsc.subcore_barrier()

    lane = lax.iota(jnp.int32, L)

    @pl.loop(0, NBLK)
    def _(blk):
        off = wid * EW + blk * B
        pltpu.sync_copy(src_hbm.at[pl.ds(off, B)], src_v)
        pltpu.sync_copy(dst_hbm.at[pl.ds(off, B)], dst_v)
        pltpu.sync_copy(et_hbm.at[pl.ds(off, B)], et_v)

        @pl.loop(0, B, step=L)
        def _(g):
            srcv = src_v[pl.ds(g, L)]
            dstv = dst_v[pl.ds(g, L)]
            etv = et_v[pl.ds(g, L)]
            rows = g + lane
            bs = srcv * 8
            bd = dstv * 8 + 4
            br = etv * 4
            for hh in range(H):
                a = plsc.load_gather(sctab_v, [bs + hh])
                b = plsc.load_gather(sctab_v, [bd + hh])
                c = plsc.load_gather(srel_v, [br + hh])
                al = a + b + c
                al = jnp.maximum(al, al * 0.2)
                ev = jnp.exp(al)
                plsc.store_scatter(ex_v, [rows * 4 + hh], ev)
                plsc.store_scatter(
                    expad_v, [rows, jnp.full((L,), hh, jnp.int32)], ev)

        pltpu.sync_copy(ex_v, ex_hbm.at[pl.ds(off * 4, B * 4)])
        pltpu.sync_copy(expad_v, dacc_sh.at[dst_v], add=True)

    plsc.subcore_barrier()

    pltpu.sync_copy(dacc_sh.at[pl.ds(sid * NR, NR)],
                    dpart_hbm.at[cid, pl.ds(sid * NR, NR)])


@jax.jit
def _k2(src, dst, et, sctab, srel_pad):
    kfn = pl.kernel(
        _k2_body,
        out_type=(_f32((E * 4,)), _f32((NC, N, 16))),
        mesh=_mesh,
        scratch_types=[
            pltpu.VMEM((N * 8,), jnp.float32),
            pltpu.VMEM((RPAD * 4,), jnp.float32),
            pltpu.VMEM((B,), jnp.int32),
            pltpu.VMEM((B,), jnp.int32),
            pltpu.VMEM((B,), jnp.int32),
            pltpu.VMEM((B * 4,), jnp.float32),
            pltpu.VMEM((B, 16), jnp.float32),
            pltpu.VMEM((ZR, 16), jnp.float32),
            pltpu.VMEM_SHARED((N, 16), jnp.float32),
        ],
        compiler_params=_cp,
    )
    return kfn(src, dst, et, sctab.reshape(N * 8), srel_pad.reshape(RPAD * 4))


def _k3_body(dp_ref, invd_ref):
    d = dp_ref[0] + dp_ref[1]
    iv = 1.0 / (d + 1e-16)
    col = lax.broadcasted_iota(jnp.int32, (N, 16), 1)
    invd_ref[...] = jnp.where(col < H, iv, 0.0)


_k3 = pl.pallas_call(_k3_body, out_shape=_f32((N, 16)))


def _k4_body(src_hbm, dst_hbm, et_hbm, ex_hbm, invd_hbm, h_hbm, rel_hbm,
             attn_hbm, opart_hbm,
             rel_tab,
             src0, dst0, dsc0, et0, ex0, hsrc0, ivd0, attn0, msg0,
             src1, dst1, dsc1, et1, ex1, hsrc1, ivd1, attn1, msg1,
             zbuf_v, sem_in0, sem_in1, sem_g0, sem_g1,
             sem_a0, sem_a1, sem_s0, sem_s1, oacc_sh):
    cid = lax.axis_index("c")
    sid = lax.axis_index("s")
    wid = cid * NS + sid

    srcs = (src0, src1)
    dsts = (dst0, dst1)
    dscs = (dsc0, dsc1)
    ets = (et0, et1)
    exs = (ex0, ex1)
    hsrcs = (hsrc0, hsrc1)
    ivds = (ivd0, ivd1)
    attns = (attn0, attn1)
    msgs = (msg0, msg1)
    sem_in = (sem_in0, sem_in1)
    sem_g = (sem_g0, sem_g1)
    sem_a = (sem_a0, sem_a1)
    sem_s = (sem_s0, sem_s1)

    pltpu.sync_copy(rel_hbm, rel_tab)

    zeros16 = jnp.zeros((L,), jnp.float32)

    @pl.loop(0, ZR)
    def _(r):
        for cc in range(D // L):
            zbuf_v[r, pl.ds(cc * L, L)] = zeros16

    @pl.loop(0, NZ)
    def _(k):
        pltpu.sync_copy(zbuf_v, oacc_sh.at[pl.ds(sid * NR + k * ZR, ZR)])

    plsc.subcore_barrier()

    lane = lax.iota(jnp.int32, L)
    lane_d4 = lax.shift_right_logical(lane, 2)
    lane_m4 = jnp.bitwise_and(lane, 3)

    def in_copies(blk, p):
        off = wid * EW + blk * B4
        return (
            pltpu.make_async_copy(src_hbm.at[pl.ds(off, B4)], srcs[p],
                                  sem_in[p]),
            pltpu.make_async_copy(dst_hbm.at[pl.ds(off, B4)], dsts[p],
                                  sem_in[p]),
            pltpu.make_async_copy(et_hbm.at[pl.ds(off, B4)], ets[p],
                                  sem_in[p]),
            pltpu.make_async_copy(ex_hbm.at[pl.ds(off * 4, B4 * 4)], exs[p],
                                  sem_in[p]),
        )

    def g_copies(p):
        return (
            pltpu.make_async_copy(h_hbm.at[srcs[p]], hsrcs[p], sem_g[p]),
            pltpu.make_async_copy(invd_hbm.at[dsts[p]], ivds[p], sem_g[p]),
        )

    def attn_copy(blk, p):
        off = wid * EW + blk * B4
        return pltpu.make_async_copy(
            attns[p], attn_hbm.at[pl.ds(off * 4, B4 * 4)], sem_a[p])

    def issue_in(blk, p):
        for c in in_copies(blk, p):
            c.start()

    def wait_in(blk, p):
        for c in in_copies(blk, p):
            c.wait()

    def issue_g(p):
        for c in g_copies(p):
            c.start()

    def wait_g(p):
        for c in g_copies(p):
            c.wait()

    def drain_out(blk, p):
        @pl.when(blk >= 2)
        def _():
            attn_copy(blk - 2, p).wait()
            pltpu.make_async_copy(msgs[p], oacc_sh.at[dscs[p]],
                                  sem_s[p]).wait()

    def compute(blk, p):
        drain_out(blk, p)

        @pl.loop(0, B4 // 4)
        def _(g):
            iv = plsc.load_gather(ivds[p], [g * 4 + lane_d4, lane_m4])
            exv = exs[p][pl.ds(g * L, L)]
            attns[p][pl.ds(g * L, L)] = exv * iv

        attn_copy(blk, p).start()

        @pl.loop(0, B4 // 4)
        def _(g):
            at16 = attns[p][pl.ds(g * L, L)]
            et4 = plsc.load_gather(ets[p], [g * 4 + lane_d4])
            for el in range(4):
                e = g * 4 + el
                rb = jnp.broadcast_to(et4[el * 4], (L,)) * D + lane
                for hh in range(H):
                    sp = jnp.broadcast_to(at16[el * 4 + hh], (L,))
                    for cc in range(2):
                        col = hh * HD + cc * L
                        hv = hsrcs[p][e, pl.ds(col, L)]
                        rv = plsc.load_gather(rel_tab, [rb + col])
                        msgs[p][e, pl.ds(col, L)] = (hv + rv) * sp

        for o in (0, 16, B4 - L):
            dscs[p][pl.ds(o, L)] = dsts[p][pl.ds(o, L)]
        pltpu.async_copy(msgs[p], oacc_sh.at[dscs[p]], sem_s[p], add=True)

    issue_in(0, 0)
    issue_in(1, 1)
    wait_in(0, 0)
    issue_g(0)

    @pl.loop(0, NBLK4 // 2)
    def _(it):
        e0 = it * 2
        wait_in(e0 + 1, 1)
        issue_g(1)
        wait_g(0)
        compute(e0, 0)

        @pl.when(e0 + 2 < NBLK4)
        def _():
            issue_in(e0 + 2, 0)
            wait_in(e0 + 2, 0)
            issue_g(0)

        wait_g(1)
        compute(e0 + 1, 1)

        @pl.when(e0 + 3 < NBLK4)
        def _():
            issue_in(e0 + 3, 1)

    attn_copy(NBLK4 - 2, 0).wait()
    pltpu.make_async_copy(msgs[0], oacc_sh.at[dscs[0]], sem_s[0]).wait()
    attn_copy(NBLK4 - 1, 1).wait()
    pltpu.make_async_copy(msgs[1], oacc_sh.at[dscs[1]], sem_s[1]).wait()

    plsc.subcore_barrier()

    pltpu.sync_copy(oacc_sh.at[pl.ds(sid * NR, NR)],
                    opart_hbm.at[cid, pl.ds(sid * NR, NR)])


@jax.jit
def _k4(src, dst, et, ex, invd, h, rel_emb):
    buf_set = [
        pltpu.VMEM((B4,), jnp.int32),
        pltpu.VMEM((B4,), jnp.int32),
        pltpu.VMEM((B4,), jnp.int32),
        pltpu.VMEM((B4,), jnp.int32),
        pltpu.VMEM((B4 * 4,), jnp.float32),
        pltpu.VMEM((B4, D), jnp.float32),
        pltpu.VMEM((B4, 16), jnp.float32),
        pltpu.VMEM((B4 * 4,), jnp.float32),
        pltpu.VMEM((B4, D), jnp.float32),
    ]
    kfn = pl.kernel(
        _k4_body,
        out_type=(_f32((E * 4,)), _f32((NC, N, D))),
        mesh=_mesh,
        scratch_types=(
            [pltpu.VMEM((R * D,), jnp.float32)]
            + buf_set + buf_set
            + [
                pltpu.VMEM((ZR, D), jnp.float32),
                pltpu.SemaphoreType.DMA,
                pltpu.SemaphoreType.DMA,
                pltpu.SemaphoreType.DMA,
                pltpu.SemaphoreType.DMA,
                pltpu.SemaphoreType.DMA,
                pltpu.SemaphoreType.DMA,
                pltpu.SemaphoreType.DMA,
                pltpu.SemaphoreType.DMA,
                pltpu.VMEM_SHARED((N, D), jnp.float32),
            ]
        ),
        compiler_params=_cp,
    )
    return kfn(src, dst, et, ex, invd, h, rel_emb.reshape(R * D))


def _k5_body(op_ref, out_ref):
    out_ref[...] = jax.nn.gelu(op_ref[0] + op_ref[1])


_k5 = pl.pallas_call(_k5_body, out_shape=_f32((N, D)))


def _blockdiag(att):
    d = jnp.arange(D)
    return jnp.zeros((D, H), jnp.float32).at[d, d // HD].set(att.reshape(D))


@jax.jit
def kernel(x, edge_index, edge_type, W, rel_emb, att_src, att_dst, att_rel):
    src = edge_index[0]
    dst = edge_index[1]
    et = edge_type

    asrc = _blockdiag(att_src)
    adst = _blockdiag(att_dst)
    arel = _blockdiag(att_rel)

    h, sctab, srel = _k1(x, W, asrc, adst, rel_emb, arel)
    srel_pad = jnp.pad(srel, ((0, RPAD - R), (0, 0)))

    ex, dpart = _k2(src, dst, et, sctab, srel_pad)
    invd = _k3(dpart)
    attn_flat, opart = _k4(src, dst, et, ex, invd, h, rel_emb)
    out = _k5(opart)
    return out, attn_flat.reshape(E, H)


# speedup vs baseline: 1.0480x; 1.0013x over previous
import dataclasses
import functools

import jax
import jax.numpy as jnp
from jax import lax
from jax.experimental import pallas as pl
from jax.experimental.pallas import tpu as pltpu
from jax.experimental.pallas import tpu_sc as plsc

N, E, D, R, H = 10000, 320000, 128, 38, 4
HD = D // H
RPAD = 40

NC, NS, L = 2, 16, 16
NW = NC * NS
EW = E // NW
B = 80
NBLK = EW // B
B4 = 40
NBLK4 = EW // B4
NR = N // NS
ZR = 25
NZ = NR // ZR

_mesh = plsc.VectorSubcoreMesh(core_axis_name="c", subcore_axis_name="s")

_cp = pltpu.CompilerParams()
_fields = pltpu.CompilerParams.__dataclass_fields__
if "needs_layout_passes" in _fields:
    _cp = dataclasses.replace(_cp, needs_layout_passes=False)
if "use_tc_tiling_on_sc" in _fields:
    _cp = dataclasses.replace(_cp, use_tc_tiling_on_sc=False)


def _f32(shape):
    return jax.ShapeDtypeStruct(shape, jnp.float32)


def _k1_body(x_ref, w_ref, asrc_ref, adst_ref, rel_ref, arel_ref,
             h_ref, st_ref, sr_ref):
    h = jnp.dot(x_ref[...], w_ref[...], preferred_element_type=jnp.float32)
    h_ref[...] = h
    ss = jnp.dot(h, asrc_ref[...], preferred_element_type=jnp.float32)
    sd = jnp.dot(h, adst_ref[...], preferred_element_type=jnp.float32)
    st_ref[...] = jnp.concatenate([ss, sd], axis=1)
    sr_ref[...] = jnp.dot(rel_ref[...], arel_ref[...],
                          preferred_element_type=jnp.float32)


_k1 = pl.pallas_call(
    _k1_body,
    out_shape=(_f32((N, D)), _f32((N, 8)), _f32((R, H))),
)


def _k2_body(src_hbm, dst_hbm, et_hbm, sctab_hbm, srel_hbm,
             ex_hbm, dpart_hbm,
             sctab_v, srel_v, src_v, dst_v, et_v, ex_v, expad_v, zbuf_v,
             dacc_sh):
    cid = lax.axis_index("c")
    sid = lax.axis_index("s")
    wid = cid * NS + sid

    pltpu.sync_copy(sctab_hbm, sctab_v)
    pltpu.sync_copy(srel_hbm, srel_v)

    zeros16 = jnp.zeros((L,), jnp.float32)

    @pl.loop(0, B)
    def _(r):
        expad_v[r, pl.ds(0, L)] = zeros16

    @pl.loop(0, ZR)
    def _(r):
        zbuf_v[r, pl.ds(0, L)] = zeros16

    @pl.loop(0, NZ)
    def _(k):
        pltpu.sync_copy(zbuf_v, dacc_sh.at[pl.ds(sid * NR + k * ZR, ZR)])

    plsc.subcore_barrier()

    lane = lax.iota(jnp.int32, L)

    @pl.loop(0, NBLK)
    def _(blk):
        off = wid * EW + blk * B
        pltpu.sync_copy(src_hbm.at[pl.ds(off, B)], src_v)
        pltpu.sync_copy(dst_hbm.at[pl.ds(off, B)], dst_v)
        pltpu.sync_copy(et_hbm.at[pl.ds(off, B)], et_v)

        @pl.loop(0, B, step=L)
        def _(g):
            srcv = src_v[pl.ds(g, L)]
            dstv = dst_v[pl.ds(g, L)]
            etv = et_v[pl.ds(g, L)]
            rows = g + lane
            bs = srcv * 8
            bd = dstv * 8 + 4
            br = etv * 4
            for hh in range(H):
                a = plsc.load_gather(sctab_v, [bs + hh])
                b = plsc.load_gather(sctab_v, [bd + hh])
                c = plsc.load_gather(srel_v, [br + hh])
                al = a + b + c
                al = jnp.maximum(al, al * 0.2)
                ev = jnp.exp(al)
                plsc.store_scatter(ex_v, [rows * 4 + hh], ev)
                plsc.store_scatter(
                    expad_v, [rows, jnp.full((L,), hh, jnp.int32)], ev)

        pltpu.sync_copy(ex_v, ex_hbm.at[pl.ds(off * 4, B * 4)])
        pltpu.sync_copy(expad_v, dacc_sh.at[dst_v], add=True)

    plsc.subcore_barrier()

    pltpu.sync_copy(dacc_sh.at[pl.ds(sid * NR, NR)],
                    dpart_hbm.at[cid, pl.ds(sid * NR, NR)])


@jax.jit
def _k2(src, dst, et, sctab, srel_pad):
    kfn = pl.kernel(
        _k2_body,
        out_type=(_f32((E * 4,)), _f32((NC, N, 16))),
        mesh=_mesh,
        scratch_types=[
            pltpu.VMEM((N * 8,), jnp.float32),
            pltpu.VMEM((RPAD * 4,), jnp.float32),
            pltpu.VMEM((B,), jnp.int32),
            pltpu.VMEM((B,), jnp.int32),
            pltpu.VMEM((B,), jnp.int32),
            pltpu.VMEM((B * 4,), jnp.float32),
            pltpu.VMEM((B, 16), jnp.float32),
            pltpu.VMEM((ZR, 16), jnp.float32),
            pltpu.VMEM_SHARED((N, 16), jnp.float32),
        ],
        compiler_params=_cp,
    )
    return kfn(src, dst, et, sctab.reshape(N * 8), srel_pad.reshape(RPAD * 4))


def _k3_body(dp_ref, invd_ref):
    d = dp_ref[0] + dp_ref[1]
    iv = 1.0 / (d + 1e-16)
    col = lax.broadcasted_iota(jnp.int32, (N, 16), 1)
    invd_ref[...] = jnp.where(col < H, iv, 0.0)


_k3 = pl.pallas_call(_k3_body, out_shape=_f32((N, 16)))


def _k4_body(src_hbm, dst_hbm, et_hbm, ex_hbm, dpart_hbm, h_hbm, rel_hbm,
             attn_hbm, opart_hbm, invd_hbm,
             rel_tab, dbuf0, dbuf1,
             src0, dst0, dsc0, et0, ex0, hsrc0, ivd0, attn0, msg0,
             src1, dst1, dsc1, et1, ex1, hsrc1, ivd1, attn1, msg1,
             zbuf_v, sem_in0, sem_in1, sem_g0, sem_g1,
             sem_a0, sem_a1, sem_s0, sem_s1, oacc_sh):
    cid = lax.axis_index("c")
    sid = lax.axis_index("s")
    wid = cid * NS + sid

    srcs = (src0, src1)
    dsts = (dst0, dst1)
    dscs = (dsc0, dsc1)
    ets = (et0, et1)
    exs = (ex0, ex1)
    hsrcs = (hsrc0, hsrc1)
    ivds = (ivd0, ivd1)
    attns = (attn0, attn1)
    msgs = (msg0, msg1)
    sem_in = (sem_in0, sem_in1)
    sem_g = (sem_g0, sem_g1)
    sem_a = (sem_a0, sem_a1)
    sem_s = (sem_s0, sem_s1)

    pltpu.sync_copy(rel_hbm, rel_tab)

    zeros16 = jnp.zeros((L,), jnp.float32)

    @pl.loop(0, ZR)
    def _(r):
        for cc in range(D // L):
            zbuf_v[r, pl.ds(cc * L, L)] = zeros16

    @pl.loop(0, NZ)
    def _(k):
        pltpu.sync_copy(zbuf_v, oacc_sh.at[pl.ds(sid * NR + k * ZR, ZR)])

    @pl.loop(0, NR // 125)
    def _(k):
        rbase = sid * NR + k * 125
        pltpu.sync_copy(dpart_hbm.at[0, pl.ds(rbase, 125)], dbuf0)
        pltpu.sync_copy(dpart_hbm.at[1, pl.ds(rbase, 125)], dbuf1)

        @pl.loop(0, 125)
        def _(r):
            a = dbuf0[r, pl.ds(0, L)]
            b = dbuf1[r, pl.ds(0, L)]
            dbuf0[r, pl.ds(0, L)] = 1.0 / (a + b + 1e-16)

        pltpu.sync_copy(dbuf0, invd_hbm.at[cid, pl.ds(rbase, 125)])

    plsc.subcore_barrier()

    lane = lax.iota(jnp.int32, L)
    lane_d4 = lax.shift_right_logical(lane, 2)
    lane_m4 = jnp.bitwise_and(lane, 3)

    def in_copies(blk, p):
        off = wid * EW + blk * B4
        return (
            pltpu.make_async_copy(src_hbm.at[pl.ds(off, B4)], srcs[p],
                                  sem_in[p]),
            pltpu.make_async_copy(dst_hbm.at[pl.ds(off, B4)], dsts[p],
                                  sem_in[p]),
            pltpu.make_async_copy(et_hbm.at[pl.ds(off, B4)], ets[p],
                                  sem_in[p]),
            pltpu.make_async_copy(ex_hbm.at[pl.ds(off * 4, B4 * 4)], exs[p],
                                  sem_in[p]),
        )

    def g_copies(p):
        return (
            pltpu.make_async_copy(h_hbm.at[srcs[p]], hsrcs[p], sem_g[p]),
            pltpu.make_async_copy(invd_hbm.at[cid].at[dsts[p]], ivds[p],
                                  sem_g[p]),
        )

    def attn_copy(blk, p):
        off = wid * EW + blk * B4
        return pltpu.make_async_copy(
            attns[p], attn_hbm.at[pl.ds(off * 4, B4 * 4)], sem_a[p])

    def issue_in(blk, p):
        for c in in_copies(blk, p):
            c.start()

    def wait_in(blk, p):
        for c in in_copies(blk, p):
            c.wait()

    def issue_g(p):
        for c in g_copies(p):
            c.start()

    def wait_g(p):
        for c in g_copies(p):
            c.wait()

    def drain_out(blk, p):
        @pl.when(blk >= 2)
        def _():
            attn_copy(blk - 2, p).wait()
            pltpu.make_async_copy(msgs[p], oacc_sh.at[dscs[p]],
                                  sem_s[p]).wait()

    def compute(blk, p):
        drain_out(blk, p)

        @pl.loop(0, B4 // 4)
        def _(g):
            iv = plsc.load_gather(ivds[p], [g * 4 + lane_d4, lane_m4])
            exv = exs[p][pl.ds(g * L, L)]
            attns[p][pl.ds(g * L, L)] = exv * iv

        attn_copy(blk, p).start()

        @pl.loop(0, B4 // 4)
        def _(g):
            at16 = attns[p][pl.ds(g * L, L)]
            et4 = plsc.load_gather(ets[p], [g * 4 + lane_d4])
            for el in range(4):
                e = g * 4 + el
                rb = jnp.broadcast_to(et4[el * 4], (L,)) * D + lane
                for hh in range(H):
                    sp = jnp.broadcast_to(at16[el * 4 + hh], (L,))
                    for cc in range(2):
                        col = hh * HD + cc * L
                        hv = hsrcs[p][e, pl.ds(col, L)]
                        rv = plsc.load_gather(rel_tab, [rb + col])
                        msgs[p][e, pl.ds(col, L)] = (hv + rv) * sp

        for o in (0, 16, B4 - L):
            dscs[p][pl.ds(o, L)] = dsts[p][pl.ds(o, L)]
        pltpu.async_copy(msgs[p], oacc_sh.at[dscs[p]], sem_s[p], add=True)

    issue_in(0, 0)
    issue_in(1, 1)
    wait_in(0, 0)
    issue_g(0)

    @pl.loop(0, NBLK4 // 2)
    def _(it):
        e0 = it * 2
        wait_in(e0 + 1, 1)
        issue_g(1)
        wait_g(0)
        compute(e0, 0)

        @pl.when(e0 + 2 < NBLK4)
        def _():
            issue_in(e0 + 2, 0)
            wait_in(e0 + 2, 0)
            issue_g(0)

        wait_g(1)
        compute(e0 + 1, 1)

        @pl.when(e0 + 3 < NBLK4)
        def _():
            issue_in(e0 + 3, 1)

    attn_copy(NBLK4 - 2, 0).wait()
    pltpu.make_async_copy(msgs[0], oacc_sh.at[dscs[0]], sem_s[0]).wait()
    attn_copy(NBLK4 - 1, 1).wait()
    pltpu.make_async_copy(msgs[1], oacc_sh.at[dscs[1]], sem_s[1]).wait()

    plsc.subcore_barrier()

    pltpu.sync_copy(oacc_sh.at[pl.ds(sid * NR, NR)],
                    opart_hbm.at[cid, pl.ds(sid * NR, NR)])


@jax.jit
def _k4(src, dst, et, ex, dpart, h, rel_emb):
    buf_set = [
        pltpu.VMEM((B4,), jnp.int32),
        pltpu.VMEM((B4,), jnp.int32),
        pltpu.VMEM((B4,), jnp.int32),
        pltpu.VMEM((B4,), jnp.int32),
        pltpu.VMEM((B4 * 4,), jnp.float32),
        pltpu.VMEM((B4, D), jnp.float32),
        pltpu.VMEM((B4, 16), jnp.float32),
        pltpu.VMEM((B4 * 4,), jnp.float32),
        pltpu.VMEM((B4, D), jnp.float32),
    ]
    kfn = pl.kernel(
        _k4_body,
        out_type=(_f32((E * 4,)), _f32((NC, N, D)), _f32((NC, N, 16))),
        mesh=_mesh,
        scratch_types=(
            [pltpu.VMEM((R * D,), jnp.float32),
             pltpu.VMEM((125, 16), jnp.float32),
             pltpu.VMEM((125, 16), jnp.float32)]
            + buf_set + buf_set
            + [
                pltpu.VMEM((ZR, D), jnp.float32),
                pltpu.SemaphoreType.DMA,
                pltpu.SemaphoreType.DMA,
                pltpu.SemaphoreType.DMA,
                pltpu.SemaphoreType.DMA,
                pltpu.SemaphoreType.DMA,
                pltpu.SemaphoreType.DMA,
                pltpu.SemaphoreType.DMA,
                pltpu.SemaphoreType.DMA,
                pltpu.VMEM_SHARED((N, D), jnp.float32),
            ]
        ),
        compiler_params=_cp,
    )
    attn_flat, opart, _ = kfn(src, dst, et, ex, dpart, h,
                              rel_emb.reshape(R * D))
    return attn_flat, opart


def _k5_body(op_ref, out_ref):
    out_ref[...] = jax.nn.gelu(op_ref[0] + op_ref[1])


_k5 = pl.pallas_call(_k5_body, out_shape=_f32((N, D)))


def _blockdiag(att):
    d = jnp.arange(D)
    return jnp.zeros((D, H), jnp.float32).at[d, d // HD].set(att.reshape(D))


@jax.jit
def kernel(x, edge_index, edge_type, W, rel_emb, att_src, att_dst, att_rel):
    src = edge_index[0]
    dst = edge_index[1]
    et = edge_type

    asrc = _blockdiag(att_src)
    adst = _blockdiag(att_dst)
    arel = _blockdiag(att_rel)

    h, sctab, srel = _k1(x, W, asrc, adst, rel_emb, arel)
    srel_pad = jnp.pad(srel, ((0, RPAD - R), (0, 0)))

    ex, dpart = _k2(src, dst, et, sctab, srel_pad)
    attn_flat, opart = _k4(src, dst, et, ex, dpart, h, rel_emb)
    out = _k5(opart)
    return out, attn_flat.reshape(E, H)


# speedup vs baseline: 1.0525x; 1.0044x over previous
import dataclasses
import functools

import jax
import jax.numpy as jnp
from jax import lax
from jax.experimental import pallas as pl
from jax.experimental.pallas import tpu as pltpu
from jax.experimental.pallas import tpu_sc as plsc

N, E, D, R, H = 10000, 320000, 128, 38, 4
HD = D // H
RPAD = 40

NC, NS, L = 2, 16, 16
NW = NC * NS
EW = E // NW
B = 80
NBLK = EW // B
B4 = 40
NBLK4 = EW // B4
NR = N // NS
ZR = 25
NZ = NR // ZR

_mesh = plsc.VectorSubcoreMesh(core_axis_name="c", subcore_axis_name="s")

_cp = pltpu.CompilerParams()
_fields = pltpu.CompilerParams.__dataclass_fields__
if "needs_layout_passes" in _fields:
    _cp = dataclasses.replace(_cp, needs_layout_passes=False)
if "use_tc_tiling_on_sc" in _fields:
    _cp = dataclasses.replace(_cp, use_tc_tiling_on_sc=False)


def _f32(shape):
    return jax.ShapeDtypeStruct(shape, jnp.float32)


def _k1_body(x_ref, w_ref, asrc_ref, adst_ref, rel_ref, arel_ref,
             h_ref, st_ref, sr_ref):
    h = jnp.dot(x_ref[...], w_ref[...], preferred_element_type=jnp.float32)
    h_ref[...] = h
    ss = jnp.dot(h, asrc_ref[...], preferred_element_type=jnp.float32)
    sd = jnp.dot(h, adst_ref[...], preferred_element_type=jnp.float32)
    st_ref[...] = jnp.concatenate([ss, sd], axis=1)
    sr_ref[...] = jnp.dot(rel_ref[...], arel_ref[...],
                          preferred_element_type=jnp.float32)


_k1 = pl.pallas_call(
    _k1_body,
    out_shape=(_f32((N, D)), _f32((N, 8)), _f32((R, H))),
)


def _k2_body(src_hbm, dst_hbm, et_hbm, sctab_hbm, srel_hbm,
             ex_hbm, dpart_hbm,
             sctab_v, srel_v, src_v, dst_v, et_v, ex_v, dtab_v):
    cid = lax.axis_index("c")
    sid = lax.axis_index("s")
    wid = cid * NS + sid

    pltpu.sync_copy(sctab_hbm, sctab_v)
    pltpu.sync_copy(srel_hbm, srel_v)

    zeros16 = jnp.zeros((L,), jnp.float32)

    @pl.loop(0, N * 4 // L)
    def _(i):
        dtab_v[pl.ds(i * L, L)] = zeros16

    lane = lax.iota(jnp.int32, L)

    @pl.loop(0, NBLK)
    def _(blk):
        off = wid * EW + blk * B
        pltpu.sync_copy(src_hbm.at[pl.ds(off, B)], src_v)
        pltpu.sync_copy(dst_hbm.at[pl.ds(off, B)], dst_v)
        pltpu.sync_copy(et_hbm.at[pl.ds(off, B)], et_v)

        @pl.loop(0, B, step=L)
        def _(g):
            srcv = src_v[pl.ds(g, L)]
            dstv = dst_v[pl.ds(g, L)]
            etv = et_v[pl.ds(g, L)]
            rows = g + lane
            bs = srcv * 8
            bd = dstv * 8 + 4
            br = etv * 4
            for hh in range(H):
                a = plsc.load_gather(sctab_v, [bs + hh])
                b = plsc.load_gather(sctab_v, [bd + hh])
                c = plsc.load_gather(srel_v, [br + hh])
                al = a + b + c
                al = jnp.maximum(al, al * 0.2)
                ev = jnp.exp(al)
                plsc.store_scatter(ex_v, [rows * 4 + hh], ev)
                plsc.addupdate_scatter(dtab_v, [dstv + hh * N], ev)

        pltpu.sync_copy(ex_v, ex_hbm.at[pl.ds(off * 4, B * 4)])

    pltpu.sync_copy(dtab_v, dpart_hbm.at[wid])


@jax.jit
def _k2(src, dst, et, sctab, srel_pad):
    kfn = pl.kernel(
        _k2_body,
        out_type=(_f32((E * 4,)), _f32((NW, N * 4))),
        mesh=_mesh,
        scratch_types=[
            pltpu.VMEM((N * 8,), jnp.float32),
            pltpu.VMEM((RPAD * 4,), jnp.float32),
            pltpu.VMEM((B,), jnp.int32),
            pltpu.VMEM((B,), jnp.int32),
            pltpu.VMEM((B,), jnp.int32),
            pltpu.VMEM((B * 4,), jnp.float32),
            pltpu.VMEM((N * 4,), jnp.float32),
        ],
        compiler_params=_cp,
    )
    return kfn(src, dst, et, sctab.reshape(N * 8), srel_pad.reshape(RPAD * 4))


def _k3_body(dp_ref, invd_ref):
    d = jnp.sum(dp_ref[...], axis=0)
    iv = jnp.transpose(1.0 / (d + 1e-16))
    invd_ref[...] = jnp.concatenate(
        [iv, jnp.zeros((N, 12), jnp.float32)], axis=1)


_k3 = pl.pallas_call(_k3_body, out_shape=_f32((N, 16)))


def _k4_body(src_hbm, dst_hbm, et_hbm, ex_hbm, invd_hbm, h_hbm, rel_hbm,
             attn_hbm, opart_hbm,
             rel_tab,
             src0, dst0, dsc0, et0, ex0, hsrc0, ivd0, attn0, msg0,
             src1, dst1, dsc1, et1, ex1, hsrc1, ivd1, attn1, msg1,
             zbuf_v, sem_in0, sem_in1, sem_g0, sem_g1,
             sem_a0, sem_a1, sem_s0, sem_s1, oacc_sh):
    cid = lax.axis_index("c")
    sid = lax.axis_index("s")
    wid = cid * NS + sid

    srcs = (src0, src1)
    dsts = (dst0, dst1)
    dscs = (dsc0, dsc1)
    ets = (et0, et1)
    exs = (ex0, ex1)
    hsrcs = (hsrc0, hsrc1)
    ivds = (ivd0, ivd1)
    attns = (attn0, attn1)
    msgs = (msg0, msg1)
    sem_in = (sem_in0, sem_in1)
    sem_g = (sem_g0, sem_g1)
    sem_a = (sem_a0, sem_a1)
    sem_s = (sem_s0, sem_s1)

    pltpu.sync_copy(rel_hbm, rel_tab)

    zeros16 = jnp.zeros((L,), jnp.float32)

    @pl.loop(0, ZR)
    def _(r):
        for cc in range(D // L):
            zbuf_v[r, pl.ds(cc * L, L)] = zeros16

    @pl.loop(0, NZ)
    def _(k):
        pltpu.sync_copy(zbuf_v, oacc_sh.at[pl.ds(sid * NR + k * ZR, ZR)])

    plsc.subcore_barrier()

    lane = lax.iota(jnp.int32, L)
    lane_d4 = lax.shift_right_logical(lane, 2)
    lane_m4 = jnp.bitwise_and(lane, 3)

    def in_copies(blk, p):
        off = wid * EW + blk * B4
        return (
            pltpu.make_async_copy(src_hbm.at[pl.ds(off, B4)], srcs[p],
                                  sem_in[p]),
            pltpu.make_async_copy(dst_hbm.at[pl.ds(off, B4)], dsts[p],
                                  sem_in[p]),
            pltpu.make_async_copy(et_hbm.at[pl.ds(off, B4)], ets[p],
                                  sem_in[p]),
            pltpu.make_async_copy(ex_hbm.at[pl.ds(off * 4, B4 * 4)], exs[p],
                                  sem_in[p]),
        )

    def g_copies(p):
        return (
            pltpu.make_async_copy(h_hbm.at[srcs[p]], hsrcs[p], sem_g[p]),
            pltpu.make_async_copy(invd_hbm.at[dsts[p]], ivds[p], sem_g[p]),
        )

    def attn_copy(blk, p):
        off = wid * EW + blk * B4
        return pltpu.make_async_copy(
            attns[p], attn_hbm.at[pl.ds(off * 4, B4 * 4)], sem_a[p])

    def issue_in(blk, p):
        for c in in_copies(blk, p):
            c.start()

    def wait_in(blk, p):
        for c in in_copies(blk, p):
            c.wait()

    def issue_g(p):
        for c in g_copies(p):
            c.start()

    def wait_g(p):
        for c in g_copies(p):
            c.wait()

    def drain_out(blk, p):
        @pl.when(blk >= 2)
        def _():
            attn_copy(blk - 2, p).wait()
            pltpu.make_async_copy(msgs[p], oacc_sh.at[dscs[p]],
                                  sem_s[p]).wait()

    def compute(blk, p):
        drain_out(blk, p)

        @pl.loop(0, B4 // 4)
        def _(g):
            iv = plsc.load_gather(ivds[p], [g * 4 + lane_d4, lane_m4])
            exv = exs[p][pl.ds(g * L, L)]
            attns[p][pl.ds(g * L, L)] = exv * iv

        attn_copy(blk, p).start()

        @pl.loop(0, B4 // 4)
        def _(g):
            at16 = attns[p][pl.ds(g * L, L)]
            et4 = plsc.load_gather(ets[p], [g * 4 + lane_d4])
            for el in range(4):
                e = g * 4 + el
                rb = jnp.broadcast_to(et4[el * 4], (L,)) * D + lane
                for hh in range(H):
                    sp = jnp.broadcast_to(at16[el * 4 + hh], (L,))
                    for cc in range(2):
                        col = hh * HD + cc * L
                        hv = hsrcs[p][e, pl.ds(col, L)]
                        rv = plsc.load_gather(rel_tab, [rb + col])
                        msgs[p][e, pl.ds(col, L)] = (hv + rv) * sp

        for o in (0, 16, B4 - L):
            dscs[p][pl.ds(o, L)] = dsts[p][pl.ds(o, L)]
        pltpu.async_copy(msgs[p], oacc_sh.at[dscs[p]], sem_s[p], add=True)

    issue_in(0, 0)
    issue_in(1, 1)
    wait_in(0, 0)
    issue_g(0)

    @pl.loop(0, NBLK4 // 2)
    def _(it):
        e0 = it * 2
        wait_in(e0 + 1, 1)
        issue_g(1)
        wait_g(0)
        compute(e0, 0)

        @pl.when(e0 + 2 < NBLK4)
        def _():
            issue_in(e0 + 2, 0)
            wait_in(e0 + 2, 0)
            issue_g(0)

        wait_g(1)
        compute(e0 + 1, 1)

        @pl.when(e0 + 3 < NBLK4)
        def _():
            issue_in(e0 + 3, 1)

    attn_copy(NBLK4 - 2, 0).wait()
    pltpu.make_async_copy(msgs[0], oacc_sh.at[dscs[0]], sem_s[0]).wait()
    attn_copy(NBLK4 - 1, 1).wait()
    pltpu.make_async_copy(msgs[1], oacc_sh.at[dscs[1]], sem_s[1]).wait()

    plsc.subcore_barrier()

    pltpu.sync_copy(oacc_sh.at[pl.ds(sid * NR, NR)],
                    opart_hbm.at[cid, pl.ds(sid * NR, NR)])


@jax.jit
def _k4(src, dst, et, ex, invd, h, rel_emb):
    buf_set = [
        pltpu.VMEM((B4,), jnp.int32),
        pltpu.VMEM((B4,), jnp.int32),
        pltpu.VMEM((B4,), jnp.int32),
        pltpu.VMEM((B4,), jnp.int32),
        pltpu.VMEM((B4 * 4,), jnp.float32),
        pltpu.VMEM((B4, D), jnp.float32),
        pltpu.VMEM((B4, 16), jnp.float32),
        pltpu.VMEM((B4 * 4,), jnp.float32),
        pltpu.VMEM((B4, D), jnp.float32),
    ]
    kfn = pl.kernel(
        _k4_body,
        out_type=(_f32((E * 4,)), _f32((NC, N, D))),
        mesh=_mesh,
        scratch_types=(
            [pltpu.VMEM((R * D,), jnp.float32)]
            + buf_set + buf_set
            + [
                pltpu.VMEM((ZR, D), jnp.float32),
                pltpu.SemaphoreType.DMA,
                pltpu.SemaphoreType.DMA,
                pltpu.SemaphoreType.DMA,
                pltpu.SemaphoreType.DMA,
                pltpu.SemaphoreType.DMA,
                pltpu.SemaphoreType.DMA,
                pltpu.SemaphoreType.DMA,
                pltpu.SemaphoreType.DMA,
                pltpu.VMEM_SHARED((N, D), jnp.float32),
            ]
        ),
        compiler_params=_cp,
    )
    return kfn(src, dst, et, ex, invd, h, rel_emb.reshape(R * D))


def _k5_body(op_ref, out_ref):
    out_ref[...] = jax.nn.gelu(op_ref[0] + op_ref[1])


_k5 = pl.pallas_call(_k5_body, out_shape=_f32((N, D)))


def _blockdiag(att):
    d = jnp.arange(D)
    return jnp.zeros((D, H), jnp.float32).at[d, d // HD].set(att.reshape(D))


@jax.jit
def kernel(x, edge_index, edge_type, W, rel_emb, att_src, att_dst, att_rel):
    src = edge_index[0]
    dst = edge_index[1]
    et = edge_type

    asrc = _blockdiag(att_src)
    adst = _blockdiag(att_dst)
    arel = _blockdiag(att_rel)

    h, sctab, srel = _k1(x, W, asrc, adst, rel_emb, arel)
    srel_pad = jnp.pad(srel, ((0, RPAD - R), (0, 0)))

    ex, dpart = _k2(src, dst, et, sctab, srel_pad)
    invd = _k3(dpart.reshape(NW, 4, N))
    attn_flat, opart = _k4(src, dst, et, ex, invd, h, rel_emb)
    out = _k5(opart)
    return out, attn_flat.reshape(E, H)


# speedup vs baseline: 1.2055x; 1.1454x over previous
import dataclasses
import functools

import jax
import jax.numpy as jnp
from jax import lax
from jax.experimental import pallas as pl
from jax.experimental.pallas import tpu as pltpu
from jax.experimental.pallas import tpu_sc as plsc

N, E, D, R, H = 10000, 320000, 128, 38, 4
HD = D // H
RPAD = 40

NC, NS, L = 2, 16, 16
NW = NC * NS
EW = E // NW
B = 80
NBLK = EW // B
B4 = 40
NBLK4 = EW // B4
NR = N // NS
ZR = 25
NZ = NR // ZR

_mesh = plsc.VectorSubcoreMesh(core_axis_name="c", subcore_axis_name="s")

_cp = pltpu.CompilerParams()
_fields = pltpu.CompilerParams.__dataclass_fields__
if "needs_layout_passes" in _fields:
    _cp = dataclasses.replace(_cp, needs_layout_passes=False)
if "use_tc_tiling_on_sc" in _fields:
    _cp = dataclasses.replace(_cp, use_tc_tiling_on_sc=False)


def _f32(shape):
    return jax.ShapeDtypeStruct(shape, jnp.float32)


def _k1_body(x_ref, w_ref, asrc_ref, adst_ref, rel_ref, arel_ref,
             h_ref, st_ref, sr_ref):
    h = jnp.dot(x_ref[...], w_ref[...], preferred_element_type=jnp.float32)
    h_ref[...] = h
    ss = jnp.dot(h, asrc_ref[...], preferred_element_type=jnp.float32)
    sd = jnp.dot(h, adst_ref[...], preferred_element_type=jnp.float32)
    st_ref[...] = jnp.concatenate([ss, sd], axis=1)
    sr_ref[...] = jnp.dot(rel_ref[...], arel_ref[...],
                          preferred_element_type=jnp.float32)


_k1 = pl.pallas_call(
    _k1_body,
    out_shape=(_f32((N, D)), _f32((N, 8)), _f32((R, H))),
)


def _k2_body(src_hbm, dst_hbm, et_hbm, sctab_hbm, srel_hbm,
             ex_hbm, dpart_hbm,
             sctab_v, srel_v, dtab_v,
             src0, dst0, et0, ex0, src1, dst1, et1, ex1,
             sem_in0, sem_in1, sem_e0, sem_e1):
    cid = lax.axis_index("c")
    sid = lax.axis_index("s")
    wid = cid * NS + sid

    srcs = (src0, src1)
    dsts = (dst0, dst1)
    ets = (et0, et1)
    exs = (ex0, ex1)
    sem_in = (sem_in0, sem_in1)
    sem_e = (sem_e0, sem_e1)

    pltpu.sync_copy(sctab_hbm, sctab_v)
    pltpu.sync_copy(srel_hbm, srel_v)

    zeros16 = jnp.zeros((L,), jnp.float32)

    @pl.loop(0, N * 4 // L)
    def _(i):
        dtab_v[pl.ds(i * L, L)] = zeros16

    lane = lax.iota(jnp.int32, L)

    def in_copies(blk, p):
        off = wid * EW + blk * B
        return (
            pltpu.make_async_copy(src_hbm.at[pl.ds(off, B)], srcs[p],
                                  sem_in[p]),
            pltpu.make_async_copy(dst_hbm.at[pl.ds(off, B)], dsts[p],
                                  sem_in[p]),
            pltpu.make_async_copy(et_hbm.at[pl.ds(off, B)], ets[p],
                                  sem_in[p]),
        )

    def ex_copy(blk, p):
        off = wid * EW + blk * B
        return pltpu.make_async_copy(
            exs[p], ex_hbm.at[pl.ds(off * 4, B * 4)], sem_e[p])

    def issue_in(blk, p):
        for c in in_copies(blk, p):
            c.start()

    def wait_in(blk, p):
        for c in in_copies(blk, p):
            c.wait()

    def compute(blk, p):
        @pl.when(blk >= 2)
        def _():
            ex_copy(blk - 2, p).wait()

        @pl.loop(0, B, step=L)
        def _(g):
            srcv = srcs[p][pl.ds(g, L)]
            dstv = dsts[p][pl.ds(g, L)]
            etv = ets[p][pl.ds(g, L)]
            rows = g + lane
            bs = srcv * 8
            bd = dstv * 8 + 4
            br = etv * 4
            for hh in range(H):
                a = plsc.load_gather(sctab_v, [bs + hh])
                b = plsc.load_gather(sctab_v, [bd + hh])
                c = plsc.load_gather(srel_v, [br + hh])
                al = a + b + c
                al = jnp.maximum(al, al * 0.2)
                ev = jnp.exp(al)
                plsc.store_scatter(exs[p], [rows * 4 + hh], ev)
                plsc.addupdate_scatter(dtab_v, [dstv + hh * N], ev)

        ex_copy(blk, p).start()

    issue_in(0, 0)
    issue_in(1, 1)

    @pl.loop(0, (NBLK - 1) // 2)
    def _(it):
        e0 = it * 2
        wait_in(e0, 0)
        compute(e0, 0)
        issue_in(e0 + 2, 0)

        wait_in(e0 + 1, 1)
        compute(e0 + 1, 1)

        @pl.when(e0 + 3 < NBLK)
        def _():
            issue_in(e0 + 3, 1)

    wait_in(NBLK - 1, 0)
    compute(NBLK - 1, 0)

    ex_copy(NBLK - 2, 1).wait()
    ex_copy(NBLK - 1, 0).wait()

    pltpu.sync_copy(dtab_v, dpart_hbm.at[wid])


@jax.jit
def _k2(src, dst, et, sctab, srel_pad):
    kfn = pl.kernel(
        _k2_body,
        out_type=(_f32((E * 4,)), _f32((NW, N * 4))),
        mesh=_mesh,
        scratch_types=[
            pltpu.VMEM((N * 8,), jnp.float32),
            pltpu.VMEM((RPAD * 4,), jnp.float32),
            pltpu.VMEM((N * 4,), jnp.float32),
            pltpu.VMEM((B,), jnp.int32),
            pltpu.VMEM((B,), jnp.int32),
            pltpu.VMEM((B,), jnp.int32),
            pltpu.VMEM((B * 4,), jnp.float32),
            pltpu.VMEM((B,), jnp.int32),
            pltpu.VMEM((B,), jnp.int32),
            pltpu.VMEM((B,), jnp.int32),
            pltpu.VMEM((B * 4,), jnp.float32),
            pltpu.SemaphoreType.DMA,
            pltpu.SemaphoreType.DMA,
            pltpu.SemaphoreType.DMA,
            pltpu.SemaphoreType.DMA,
        ],
        compiler_params=_cp,
    )
    return kfn(src, dst, et, sctab.reshape(N * 8), srel_pad.reshape(RPAD * 4))


def _k3_body(dp_ref, invd_ref):
    d = jnp.sum(dp_ref[...], axis=0)
    iv = jnp.transpose(1.0 / (d + 1e-16))
    invd_ref[...] = jnp.concatenate(
        [iv, jnp.zeros((N, 12), jnp.float32)], axis=1)


_k3 = pl.pallas_call(_k3_body, out_shape=_f32((N, 16)))


def _k4_body(src_hbm, dst_hbm, et_hbm, ex_hbm, invd_hbm, h_hbm, rel_hbm,
             attn_hbm, opart_hbm,
             rel_tab,
             src0, dst0, dsc0, et0, ex0, hsrc0, ivd0, attn0, msg0,
             src1, dst1, dsc1, et1, ex1, hsrc1, ivd1, attn1, msg1,
             zbuf_v, sem_in0, sem_in1, sem_g0, sem_g1,
             sem_a0, sem_a1, sem_s0, sem_s1, oacc_sh):
    cid = lax.axis_index("c")
    sid = lax.axis_index("s")
    wid = cid * NS + sid

    srcs = (src0, src1)
    dsts = (dst0, dst1)
    dscs = (dsc0, dsc1)
    ets = (et0, et1)
    exs = (ex0, ex1)
    hsrcs = (hsrc0, hsrc1)
    ivds = (ivd0, ivd1)
    attns = (attn0, attn1)
    msgs = (msg0, msg1)
    sem_in = (sem_in0, sem_in1)
    sem_g = (sem_g0, sem_g1)
    sem_a = (sem_a0, sem_a1)
    sem_s = (sem_s0, sem_s1)

    pltpu.sync_copy(rel_hbm, rel_tab)

    zeros16 = jnp.zeros((L,), jnp.float32)

    @pl.loop(0, ZR)
    def _(r):
        for cc in range(D // L):
            zbuf_v[r, pl.ds(cc * L, L)] = zeros16

    @pl.loop(0, NZ)
    def _(k):
        pltpu.sync_copy(zbuf_v, oacc_sh.at[pl.ds(sid * NR + k * ZR, ZR)])

    plsc.subcore_barrier()

    lane = lax.iota(jnp.int32, L)
    lane_d4 = lax.shift_right_logical(lane, 2)
    lane_m4 = jnp.bitwise_and(lane, 3)

    def in_copies(blk, p):
        off = wid * EW + blk * B4
        return (
            pltpu.make_async_copy(src_hbm.at[pl.ds(off, B4)], srcs[p],
                                  sem_in[p]),
            pltpu.make_async_copy(dst_hbm.at[pl.ds(off, B4)], dsts[p],
                                  sem_in[p]),
            pltpu.make_async_copy(et_hbm.at[pl.ds(off, B4)], ets[p],
                                  sem_in[p]),
            pltpu.make_async_copy(ex_hbm.at[pl.ds(off * 4, B4 * 4)], exs[p],
                                  sem_in[p]),
        )

    def g_copies(p):
        return (
            pltpu.make_async_copy(h_hbm.at[srcs[p]], hsrcs[p], sem_g[p]),
            pltpu.make_async_copy(invd_hbm.at[dsts[p]], ivds[p], sem_g[p]),
        )

    def attn_copy(blk, p):
        off = wid * EW + blk * B4
        return pltpu.make_async_copy(
            attns[p], attn_hbm.at[pl.ds(off * 4, B4 * 4)], sem_a[p])

    def issue_in(blk, p):
        for c in in_copies(blk, p):
            c.start()

    def wait_in(blk, p):
        for c in in_copies(blk, p):
            c.wait()

    def issue_g(p):
        for c in g_copies(p):
            c.start()

    def wait_g(p):
        for c in g_copies(p):
            c.wait()

    def drain_out(blk, p):
        @pl.when(blk >= 2)
        def _():
            attn_copy(blk - 2, p).wait()
            pltpu.make_async_copy(msgs[p], oacc_sh.at[dscs[p]],
                                  sem_s[p]).wait()

    def compute(blk, p):
        drain_out(blk, p)

        @pl.loop(0, B4 // 4)
        def _(g):
            iv = plsc.load_gather(ivds[p], [g * 4 + lane_d4, lane_m4])
            exv = exs[p][pl.ds(g * L, L)]
            attns[p][pl.ds(g * L, L)] = exv * iv

        attn_copy(blk, p).start()

        @pl.loop(0, B4 // 4)
        def _(g):
            at16 = attns[p][pl.ds(g * L, L)]
            et4 = plsc.load_gather(ets[p], [g * 4 + lane_d4])
            for el in range(4):
                e = g * 4 + el
                rb = jnp.broadcast_to(et4[el * 4], (L,)) * D + lane
                for hh in range(H):
                    sp = jnp.broadcast_to(at16[el * 4 + hh], (L,))
                    for cc in range(2):
                        col = hh * HD + cc * L
                        hv = hsrcs[p][e, pl.ds(col, L)]
                        rv = plsc.load_gather(rel_tab, [rb + col])
                        msgs[p][e, pl.ds(col, L)] = (hv + rv) * sp

        for o in (0, 16, B4 - L):
            dscs[p][pl.ds(o, L)] = dsts[p][pl.ds(o, L)]
        pltpu.async_copy(msgs[p], oacc_sh.at[dscs[p]], sem_s[p], add=True)

    issue_in(0, 0)
    issue_in(1, 1)
    wait_in(0, 0)
    issue_g(0)

    @pl.loop(0, NBLK4 // 2)
    def _(it):
        e0 = it * 2
        wait_in(e0 + 1, 1)
        issue_g(1)
        wait_g(0)
        compute(e0, 0)

        @pl.when(e0 + 2 < NBLK4)
        def _():
            issue_in(e0 + 2, 0)
            wait_in(e0 + 2, 0)
            issue_g(0)

        wait_g(1)
        compute(e0 + 1, 1)

        @pl.when(e0 + 3 < NBLK4)
        def _():
            issue_in(e0 + 3, 1)

    attn_copy(NBLK4 - 2, 0).wait()
    pltpu.make_async_copy(msgs[0], oacc_sh.at[dscs[0]], sem_s[0]).wait()
    attn_copy(NBLK4 - 1, 1).wait()
    pltpu.make_async_copy(msgs[1], oacc_sh.at[dscs[1]], sem_s[1]).wait()

    plsc.subcore_barrier()

    pltpu.sync_copy(oacc_sh.at[pl.ds(sid * NR, NR)],
                    opart_hbm.at[cid, pl.ds(sid * NR, NR)])


@jax.jit
def _k4(src, dst, et, ex, invd, h, rel_emb):
    buf_set = [
        pltpu.VMEM((B4,), jnp.int32),
        pltpu.VMEM((B4,), jnp.int32),
        pltpu.VMEM((B4,), jnp.int32),
        pltpu.VMEM((B4,), jnp.int32),
        pltpu.VMEM((B4 * 4,), jnp.float32),
        pltpu.VMEM((B4, D), jnp.float32),
        pltpu.VMEM((B4, 16), jnp.float32),
        pltpu.VMEM((B4 * 4,), jnp.float32),
        pltpu.VMEM((B4, D), jnp.float32),
    ]
    kfn = pl.kernel(
        _k4_body,
        out_type=(_f32((E * 4,)), _f32((NC, N, D))),
        mesh=_mesh,
        scratch_types=(
            [pltpu.VMEM((R * D,), jnp.float32)]
            + buf_set + buf_set
            + [
                pltpu.VMEM((ZR, D), jnp.float32),
                pltpu.SemaphoreType.DMA,
                pltpu.SemaphoreType.DMA,
                pltpu.SemaphoreType.DMA,
                pltpu.SemaphoreType.DMA,
                pltpu.SemaphoreType.DMA,
                pltpu.SemaphoreType.DMA,
                pltpu.SemaphoreType.DMA,
                pltpu.SemaphoreType.DMA,
                pltpu.VMEM_SHARED((N, D), jnp.float32),
            ]
        ),
        compiler_params=_cp,
    )
    return kfn(src, dst, et, ex, invd, h, rel_emb.reshape(R * D))


def _k5_body(op_ref, out_ref):
    out_ref[...] = jax.nn.gelu(op_ref[0] + op_ref[1])


_k5 = pl.pallas_call(_k5_body, out_shape=_f32((N, D)))


def _blockdiag(att):
    d = jnp.arange(D)
    return jnp.zeros((D, H), jnp.float32).at[d, d // HD].set(att.reshape(D))


@jax.jit
def kernel(x, edge_index, edge_type, W, rel_emb, att_src, att_dst, att_rel):
    src = edge_index[0]
    dst = edge_index[1]
    et = edge_type

    asrc = _blockdiag(att_src)
    adst = _blockdiag(att_dst)
    arel = _blockdiag(att_rel)

    h, sctab, srel = _k1(x, W, asrc, adst, rel_emb, arel)
    srel_pad = jnp.pad(srel, ((0, RPAD - R), (0, 0)))

    ex, dpart = _k2(src, dst, et, sctab, srel_pad)
    invd = _k3(dpart.reshape(NW, 4, N))
    attn_flat, opart = _k4(src, dst, et, ex, invd, h, rel_emb)
    out = _k5(opart)
    return out, attn_flat.reshape(E, H)


# speedup vs baseline: 1.2056x; 1.0001x over previous
import dataclasses

import jax
import jax.numpy as jnp
from jax import lax
from jax.experimental import pallas as pl
from jax.experimental.pallas import tpu as pltpu
from jax.experimental.pallas import tpu_sc as plsc

N, E, D, R, H = 10000, 320000, 128, 38, 4
HD = D // H
RPAD = 40

NC, NS, L = 2, 16, 16
NW = NC * NS
EW = E // NW
B = 80
NBLK = EW // B
B4 = 40
NBLK4 = EW // B4
NR = N // NS
ZR = 25
NZ = NR // ZR

_mesh = plsc.VectorSubcoreMesh(core_axis_name="c", subcore_axis_name="s")

_cp = pltpu.CompilerParams()
_fields = pltpu.CompilerParams.__dataclass_fields__
if "needs_layout_passes" in _fields:
    _cp = dataclasses.replace(_cp, needs_layout_passes=False)
if "use_tc_tiling_on_sc" in _fields:
    _cp = dataclasses.replace(_cp, use_tc_tiling_on_sc=False)


def _f32(shape):
    return jax.ShapeDtypeStruct(shape, jnp.float32)


def _k1_body(x_ref, w_ref, asrc_ref, adst_ref, rel_ref, arel_ref,
             h_ref, st_ref, sr_ref):
    h = jnp.dot(x_ref[...], w_ref[...], preferred_element_type=jnp.float32)
    h_ref[...] = h
    ss = jnp.dot(h, asrc_ref[...], preferred_element_type=jnp.float32)
    sd = jnp.dot(h, adst_ref[...], preferred_element_type=jnp.float32)
    st_ref[...] = jnp.concatenate([ss, sd], axis=1)
    sr_ref[...] = jnp.dot(rel_ref[...], arel_ref[...],
                          preferred_element_type=jnp.float32)


_k1 = pl.pallas_call(
    _k1_body,
    out_shape=(_f32((N, D)), _f32((N, 8)), _f32((R, H))),
)


def _k2_body(src_hbm, dst_hbm, et_hbm, sctab_hbm, srel_hbm,
             ex_hbm, dpart_hbm,
             sctab_v, srel_v, dtab_v,
             src0, dst0, et0, ex0, src1, dst1, et1, ex1,
             sem_in0, sem_in1, sem_e0, sem_e1):
    cid = lax.axis_index("c")
    sid = lax.axis_index("s")
    wid = cid * NS + sid

    srcs = (src0, src1)
    dsts = (dst0, dst1)
    ets = (et0, et1)
    exs = (ex0, ex1)
    sem_in = (sem_in0, sem_in1)
    sem_e = (sem_e0, sem_e1)

    pltpu.sync_copy(sctab_hbm, sctab_v)
    pltpu.sync_copy(srel_hbm, srel_v)

    zeros16 = jnp.zeros((L,), jnp.float32)

    @pl.loop(0, N * 4 // L)
    def _(i):
        dtab_v[pl.ds(i * L, L)] = zeros16

    lane = lax.iota(jnp.int32, L)

    def in_copies(blk, p):
        off = wid * EW + blk * B
        return (
            pltpu.make_async_copy(src_hbm.at[pl.ds(off, B)], srcs[p],
                                  sem_in[p]),
            pltpu.make_async_copy(dst_hbm.at[pl.ds(off, B)], dsts[p],
                                  sem_in[p]),
            pltpu.make_async_copy(et_hbm.at[pl.ds(off, B)], ets[p],
                                  sem_in[p]),
        )

    def ex_copy(blk, p):
        off = wid * EW + blk * B
        return pltpu.make_async_copy(
            exs[p], ex_hbm.at[pl.ds(off * 4, B * 4)], sem_e[p])

    def issue_in(blk, p):
        for c in in_copies(blk, p):
            c.start()

    def wait_in(blk, p):
        for c in in_copies(blk, p):
            c.wait()

    def compute(blk, p):
        @pl.when(blk >= 2)
        def _():
            ex_copy(blk - 2, p).wait()

        @pl.loop(0, B, step=L)
        def _(g):
            srcv = srcs[p][pl.ds(g, L)]
            dstv = dsts[p][pl.ds(g, L)]
            etv = ets[p][pl.ds(g, L)]
            rows = g + lane
            bs = srcv * 8
            bd = dstv * 8 + 4
            br = etv * 4
            for hh in range(H):
                a = plsc.load_gather(sctab_v, [bs + hh])
                b = plsc.load_gather(sctab_v, [bd + hh])
                c = plsc.load_gather(srel_v, [br + hh])
                al = a + b + c
                al = jnp.maximum(al, al * 0.2)
                ev = jnp.exp(al)
                plsc.store_scatter(exs[p], [rows * 4 + hh], ev)
                plsc.addupdate_scatter(dtab_v, [dstv + hh * N], ev)

        ex_copy(blk, p).start()

    issue_in(0, 0)
    issue_in(1, 1)

    @pl.loop(0, (NBLK - 1) // 2)
    def _(it):
        e0 = it * 2
        wait_in(e0, 0)
        compute(e0, 0)
        issue_in(e0 + 2, 0)

        wait_in(e0 + 1, 1)
        compute(e0 + 1, 1)

        @pl.when(e0 + 3 < NBLK)
        def _():
            issue_in(e0 + 3, 1)

    wait_in(NBLK - 1, 0)
    compute(NBLK - 1, 0)

    ex_copy(NBLK - 2, 1).wait()
    ex_copy(NBLK - 1, 0).wait()

    pltpu.sync_copy(dtab_v, dpart_hbm.at[wid])


@jax.jit
def _k2(src, dst, et, sctab, srel_pad):
    kfn = pl.kernel(
        _k2_body,
        out_type=(_f32((E * 4,)), _f32((NW, N * 4))),
        mesh=_mesh,
        scratch_types=[
            pltpu.VMEM((N * 8,), jnp.float32),
            pltpu.VMEM((RPAD * 4,), jnp.float32),
            pltpu.VMEM((N * 4,), jnp.float32),
            pltpu.VMEM((B,), jnp.int32),
            pltpu.VMEM((B,), jnp.int32),
            pltpu.VMEM((B,), jnp.int32),
            pltpu.VMEM((B * 4,), jnp.float32),
            pltpu.VMEM((B,), jnp.int32),
            pltpu.VMEM((B,), jnp.int32),
            pltpu.VMEM((B,), jnp.int32),
            pltpu.VMEM((B * 4,), jnp.float32),
            pltpu.SemaphoreType.DMA,
            pltpu.SemaphoreType.DMA,
            pltpu.SemaphoreType.DMA,
            pltpu.SemaphoreType.DMA,
        ],
        compiler_params=_cp,
    )
    return kfn(src, dst, et, sctab.reshape(N * 8), srel_pad.reshape(RPAD * 4))


def _k3_body(dp_ref, invd_ref):
    d = jnp.sum(dp_ref[...], axis=0)
    iv = jnp.transpose(1.0 / (d + 1e-16))
    invd_ref[...] = jnp.concatenate(
        [iv, jnp.zeros((N, 12), jnp.float32)], axis=1)


_k3 = pl.pallas_call(_k3_body, out_shape=_f32((N, 16)))


def _k4_body(src_hbm, dst_hbm, et_hbm, ex_hbm, invd_hbm, h_hbm, rel_hbm,
             attn_hbm, opart_hbm,
             rel_tab,
             src0, dst0, dsc0, et0, ex0, hsrc0, ivd0, attn0, msg0,
             src1, dst1, dsc1, et1, ex1, hsrc1, ivd1, attn1, msg1,
             zbuf_v, sem_in0, sem_in1, sem_g0, sem_g1,
             sem_a0, sem_a1, sem_s0, sem_s1, oacc_sh):
    cid = lax.axis_index("c")
    sid = lax.axis_index("s")
    wid = cid * NS + sid

    srcs = (src0, src1)
    dsts = (dst0, dst1)
    dscs = (dsc0, dsc1)
    ets = (et0, et1)
    exs = (ex0, ex1)
    hsrcs = (hsrc0, hsrc1)
    ivds = (ivd0, ivd1)
    attns = (attn0, attn1)
    msgs = (msg0, msg1)
    sem_in = (sem_in0, sem_in1)
    sem_g = (sem_g0, sem_g1)
    sem_a = (sem_a0, sem_a1)
    sem_s = (sem_s0, sem_s1)

    pltpu.sync_copy(rel_hbm, rel_tab)

    zeros16 = jnp.zeros((L,), jnp.float32)

    @pl.loop(0, ZR)
    def _(r):
        for cc in range(D // L):
            zbuf_v[r, pl.ds(cc * L, L)] = zeros16

    @pl.loop(0, NZ)
    def _(k):
        pltpu.sync_copy(zbuf_v, oacc_sh.at[pl.ds(sid * NR + k * ZR, ZR)])

    plsc.subcore_barrier()

    lane = lax.iota(jnp.int32, L)
    lane_d4 = lax.shift_right_logical(lane, 2)
    lane_m4 = jnp.bitwise_and(lane, 3)

    def in_copies(blk, p):
        off = wid * EW + blk * B4
        return (
            pltpu.make_async_copy(src_hbm.at[pl.ds(off, B4)], srcs[p],
                                  sem_in[p]),
            pltpu.make_async_copy(dst_hbm.at[pl.ds(off, B4)], dsts[p],
                                  sem_in[p]),
            pltpu.make_async_copy(et_hbm.at[pl.ds(off, B4)], ets[p],
                                  sem_in[p]),
            pltpu.make_async_copy(ex_hbm.at[pl.ds(off * 4, B4 * 4)], exs[p],
                                  sem_in[p]),
        )

    def g_copies(p):
        return (
            pltpu.make_async_copy(h_hbm.at[srcs[p]], hsrcs[p], sem_g[p]),
            pltpu.make_async_copy(invd_hbm.at[dsts[p]], ivds[p], sem_g[p]),
        )

    def attn_copy(blk, p):
        off = wid * EW + blk * B4
        return pltpu.make_async_copy(
            attns[p], attn_hbm.at[pl.ds(off * 4, B4 * 4)], sem_a[p])

    def issue_in(blk, p):
        for c in in_copies(blk, p):
            c.start()

    def wait_in(blk, p):
        for c in in_copies(blk, p):
            c.wait()

    def issue_g(p):
        for c in g_copies(p):
            c.start()

    def wait_g(p):
        for c in g_copies(p):
            c.wait()

    def drain_out(blk, p):
        @pl.when(blk >= 2)
        def _():
            attn_copy(blk - 2, p).wait()
            pltpu.make_async_copy(msgs[p], oacc_sh.at[dscs[p]],
                                  sem_s[p]).wait()

    def compute(blk, p):
        drain_out(blk, p)

        @pl.loop(0, B4 // 4)
        def _(g):
            iv = plsc.load_gather(ivds[p], [g * 4 + lane_d4, lane_m4])
            exv = exs[p][pl.ds(g * L, L)]
            attns[p][pl.ds(g * L, L)] = exv * iv

        attn_copy(blk, p).start()

        @pl.loop(0, B4 // 4)
        def _(g):
            at16 = attns[p][pl.ds(g * L, L)]
            et4 = plsc.load_gather(ets[p], [g * 4 + lane_d4])
            for el in range(4):
                e = g * 4 + el
                rb = jnp.broadcast_to(et4[el * 4], (L,)) * D + lane
                for hh in range(H):
                    sp = jnp.broadcast_to(at16[el * 4 + hh], (L,))
                    for cc in range(2):
                        col = hh * HD + cc * L
                        hv = hsrcs[p][e, pl.ds(col, L)]
                        rv = plsc.load_gather(rel_tab, [rb + col])
                        msgs[p][e, pl.ds(col, L)] = (hv + rv) * sp

        for o in (0, 16, B4 - L):
            dscs[p][pl.ds(o, L)] = dsts[p][pl.ds(o, L)]
        pltpu.async_copy(msgs[p], oacc_sh.at[dscs[p]], sem_s[p], add=True)

    issue_in(0, 0)
    issue_in(1, 1)
    wait_in(0, 0)
    issue_g(0)

    @pl.loop(0, NBLK4 // 2)
    def _(it):
        e0 = it * 2
        wait_in(e0 + 1, 1)
        issue_g(1)
        wait_g(0)
        compute(e0, 0)

        @pl.when(e0 + 2 < NBLK4)
        def _():
            issue_in(e0 + 2, 0)
            wait_in(e0 + 2, 0)
            issue_g(0)

        wait_g(1)
        compute(e0 + 1, 1)

        @pl.when(e0 + 3 < NBLK4)
        def _():
            issue_in(e0 + 3, 1)

    attn_copy(NBLK4 - 2, 0).wait()
    pltpu.make_async_copy(msgs[0], oacc_sh.at[dscs[0]], sem_s[0]).wait()
    attn_copy(NBLK4 - 1, 1).wait()
    pltpu.make_async_copy(msgs[1], oacc_sh.at[dscs[1]], sem_s[1]).wait()

    plsc.subcore_barrier()

    pltpu.sync_copy(oacc_sh.at[pl.ds(sid * NR, NR)],
                    opart_hbm.at[cid, pl.ds(sid * NR, NR)])


@jax.jit
def _k4(src, dst, et, ex, invd, h, rel_emb):
    buf_set = [
        pltpu.VMEM((B4,), jnp.int32),
        pltpu.VMEM((B4,), jnp.int32),
        pltpu.VMEM((B4,), jnp.int32),
        pltpu.VMEM((B4,), jnp.int32),
        pltpu.VMEM((B4 * 4,), jnp.float32),
        pltpu.VMEM((B4, D), jnp.float32),
        pltpu.VMEM((B4, 16), jnp.float32),
        pltpu.VMEM((B4 * 4,), jnp.float32),
        pltpu.VMEM((B4, D), jnp.float32),
    ]
    kfn = pl.kernel(
        _k4_body,
        out_type=(_f32((E * 4,)), _f32((NC, N, D))),
        mesh=_mesh,
        scratch_types=(
            [pltpu.VMEM((R * D,), jnp.float32)]
            + buf_set + buf_set
            + [
                pltpu.VMEM((ZR, D), jnp.float32),
                pltpu.SemaphoreType.DMA,
                pltpu.SemaphoreType.DMA,
                pltpu.SemaphoreType.DMA,
                pltpu.SemaphoreType.DMA,
                pltpu.SemaphoreType.DMA,
                pltpu.SemaphoreType.DMA,
                pltpu.SemaphoreType.DMA,
                pltpu.SemaphoreType.DMA,
                pltpu.VMEM_SHARED((N, D), jnp.float32),
            ]
        ),
        compiler_params=_cp,
    )
    return kfn(src, dst, et, ex, invd, h, rel_emb.reshape(R * D))


def _k5_body(op_ref, out_ref):
    out_ref[...] = jax.nn.gelu(op_ref[0] + op_ref[1])


_k5 = pl.pallas_call(_k5_body, out_shape=_f32((N, D)))


def _blockdiag(att):
    d = jnp.arange(D)
    return jnp.zeros((D, H), jnp.float32).at[d, d // HD].set(att.reshape(D))


@jax.jit
def kernel(x, edge_index, edge_type, W, rel_emb, att_src, att_dst, att_rel):
    src = edge_index[0]
    dst = edge_index[1]
    et = edge_type

    asrc = _blockdiag(att_src)
    adst = _blockdiag(att_dst)
    arel = _blockdiag(att_rel)

    h, sctab, srel = _k1(x, W, asrc, adst, rel_emb, arel)
    srel_pad = jnp.pad(srel, ((0, RPAD - R), (0, 0)))

    ex, dpart = _k2(src, dst, et, sctab, srel_pad)
    invd = _k3(dpart.reshape(NW, 4, N))
    attn_flat, opart = _k4(src, dst, et, ex, invd, h, rel_emb)
    out = _k5(opart)
    return out, attn_flat.reshape(E, H)
